# baseline (speedup 1.0000x reference)
import functools

import numpy as np
import jax
import jax.numpy as jnp
from jax.experimental import pallas as pl
from jax.experimental.pallas import tpu as pltpu
from jax.experimental.pallas import tpu_sc as plsc

_L = 2048
_DM = 1024
_H = 16
_DH = 64
_DFF = 4096
_NSTATE = 65
_FACTOR = 5
_U = min(int(_FACTOR * np.ceil(np.log(_L))), _L)
_NEG = -1e30


def _pe_table():
    pe = np.zeros((_L, _DM), np.float32)
    pos = np.arange(_L, dtype=np.float32)[:, None]
    div = np.exp(np.arange(0, _DM, 2, dtype=np.float32) * (-np.log(10000.0) / _DM))
    pe[:, 0::2] = np.sin(pos * div)
    pe[:, 1::2] = np.cos(pos * div)
    return pe


_PE = _pe_table()


def _sample_count_matrix_t(layer_idx):
    rng = jax.random.fold_in(jax.random.key(42), layer_idx)
    idx = np.asarray(jax.random.randint(rng, (_L, _U), 0, _L))
    cnt_t = np.zeros((_L, _L), np.float32)
    np.add.at(cnt_t, (idx, np.arange(_L)[:, None]), 1.0)
    return cnt_t


_CNT_T = [_sample_count_matrix_t(li) for li in range(2)]


def _mm_kern(x_ref, w_ref, b_ref, o_ref, acc_ref, *, nk, act):
    @pl.when(pl.program_id(2) == 0)
    def _():
        acc_ref[...] = jnp.zeros_like(acc_ref)

    acc_ref[...] += jnp.dot(x_ref[...], w_ref[...],
                            preferred_element_type=jnp.float32)

    @pl.when(pl.program_id(2) == nk - 1)
    def _():
        r = acc_ref[...] + b_ref[...]
        if act == "gelu":
            r = jax.nn.gelu(r)
        o_ref[...] = r


def _matmul(x, w, b, act=None, bm=256, bn=512, bk=512):
    m, k = x.shape
    n = w.shape[1]
    bm, bn, bk = min(bm, m), min(bn, n), min(bk, k)
    grid = (m // bm, n // bn, k // bk)
    return pl.pallas_call(
        functools.partial(_mm_kern, nk=grid[2], act=act),
        grid=grid,
        in_specs=[
            pl.BlockSpec((bm, bk), lambda i, j, kk: (i, kk)),
            pl.BlockSpec((bk, bn), lambda i, j, kk: (kk, j)),
            pl.BlockSpec((1, bn), lambda i, j, kk: (0, j)),
        ],
        out_specs=pl.BlockSpec((bm, bn), lambda i, j, kk: (i, j)),
        out_shape=jax.ShapeDtypeStruct((m, n), jnp.float32),
        scratch_shapes=[pltpu.VMEM((bm, bn), jnp.float32)],
    )(x, w, b.reshape(1, n))


def _qkv_kern(x_ref, w_ref, b_ref, o_ref):
    acc = jnp.dot(x_ref[...], w_ref[...],
                  preferred_element_type=jnp.float32) + b_ref[...]
    o_ref[0, 0] = acc[:, :_DH]
    o_ref[0, 1] = acc[:, _DH:]


def _qkv(x, wqkv, bqkv):
    bm = 1024
    grid = (_L // bm, 3, _H // 2)
    return pl.pallas_call(
        _qkv_kern,
        grid=grid,
        in_specs=[
            pl.BlockSpec((bm, _DM), lambda i, t, j: (i, 0)),
            pl.BlockSpec((_DM, 2 * _DH), lambda i, t, j: (0, t * (_H // 2) + j)),
            pl.BlockSpec((1, 2 * _DH), lambda i, t, j: (0, t * (_H // 2) + j)),
        ],
        out_specs=pl.BlockSpec((1, 2, bm, _DH), lambda i, t, j: (t, j, i, 0)),
        out_shape=jax.ShapeDtypeStruct((3, _H, _L, _DH), jnp.float32),
    )(x, wqkv, bqkv.reshape(1, 3 * _DM))


def _attnout_ln_kern(ctx_ref, w_ref, b_ref, res_ref, g_ref, be_ref, o_ref,
                     acc_ref, *, nk):
    kk = pl.program_id(1)

    @pl.when(kk == 0)
    def _():
        acc_ref[...] = jnp.zeros_like(acc_ref)

    c = ctx_ref[...]
    xcat = jnp.concatenate([c[0], c[1]], axis=1)
    acc_ref[...] += jnp.dot(xcat, w_ref[...], preferred_element_type=jnp.float32)

    @pl.when(kk == nk - 1)
    def _():
        t = acc_ref[...] + b_ref[...] + res_ref[...]
        mu = jnp.mean(t, axis=1, keepdims=True)
        d = t - mu
        var = jnp.mean(d * d, axis=1, keepdims=True)
        o_ref[...] = d * jax.lax.rsqrt(var + 1e-5) * g_ref[...] + be_ref[...]


def _attnout_ln(ctx, wo, bo, res, g, be):
    bm = 256
    nk = _H // 2
    return pl.pallas_call(
        functools.partial(_attnout_ln_kern, nk=nk),
        grid=(_L // bm, nk),
        in_specs=[
            pl.BlockSpec((2, bm, _DH), lambda i, kk: (kk, i, 0)),
            pl.BlockSpec((2 * _DH, _DM), lambda i, kk: (kk, 0)),
            pl.BlockSpec((1, _DM), lambda i, kk: (0, 0)),
            pl.BlockSpec((bm, _DM), lambda i, kk: (i, 0)),
            pl.BlockSpec((1, _DM), lambda i, kk: (0, 0)),
            pl.BlockSpec((1, _DM), lambda i, kk: (0, 0)),
        ],
        out_specs=pl.BlockSpec((bm, _DM), lambda i, kk: (i, 0)),
        out_shape=jax.ShapeDtypeStruct((_L, _DM), jnp.float32),
        scratch_shapes=[pltpu.VMEM((bm, _DM), jnp.float32)],
    )(ctx, wo, bo.reshape(1, _DM), res, g.reshape(1, _DM), be.reshape(1, _DM))


def _ffn1_kern(x_ref, w_ref, b_ref, o_ref):
    o_ref[...] = jax.nn.gelu(
        jnp.dot(x_ref[...], w_ref[...], preferred_element_type=jnp.float32)
        + b_ref[...])


def _ffn1(x, w1, b1):
    bm, bn = 1024, 512
    return pl.pallas_call(
        _ffn1_kern,
        grid=(_L // bm, _DFF // bn),
        in_specs=[
            pl.BlockSpec((bm, _DM), lambda i, j: (i, 0)),
            pl.BlockSpec((_DM, bn), lambda i, j: (0, j)),
            pl.BlockSpec((1, bn), lambda i, j: (0, j)),
        ],
        out_specs=pl.BlockSpec((bm, bn), lambda i, j: (i, j)),
        out_shape=jax.ShapeDtypeStruct((_L, _DFF), jnp.float32),
    )(x, w1, b1.reshape(1, _DFF))


def _ffn2_ln_kern(x_ref, w_ref, b_ref, res_ref, g_ref, be_ref, o_ref,
                  acc_ref, *, nk):
    kk = pl.program_id(1)

    @pl.when(kk == 0)
    def _():
        acc_ref[...] = jnp.zeros_like(acc_ref)

    acc_ref[...] += jnp.dot(x_ref[...], w_ref[...],
                            preferred_element_type=jnp.float32)

    @pl.when(kk == nk - 1)
    def _():
        t = acc_ref[...] + b_ref[...] + res_ref[...]
        mu = jnp.mean(t, axis=1, keepdims=True)
        d = t - mu
        var = jnp.mean(d * d, axis=1, keepdims=True)
        o_ref[...] = d * jax.lax.rsqrt(var + 1e-5) * g_ref[...] + be_ref[...]


def _ffn2_ln(h1, w2, b2, res, g, be):
    bm, bk = 512, 512
    nk = _DFF // bk
    return pl.pallas_call(
        functools.partial(_ffn2_ln_kern, nk=nk),
        grid=(_L // bm, nk),
        in_specs=[
            pl.BlockSpec((bm, bk), lambda i, kk: (i, kk)),
            pl.BlockSpec((bk, _DM), lambda i, kk: (kk, 0)),
            pl.BlockSpec((1, _DM), lambda i, kk: (0, 0)),
            pl.BlockSpec((bm, _DM), lambda i, kk: (i, 0)),
            pl.BlockSpec((1, _DM), lambda i, kk: (0, 0)),
            pl.BlockSpec((1, _DM), lambda i, kk: (0, 0)),
        ],
        out_specs=pl.BlockSpec((bm, _DM), lambda i, kk: (i, 0)),
        out_shape=jax.ShapeDtypeStruct((_L, _DM), jnp.float32),
        scratch_shapes=[pltpu.VMEM((bm, _DM), jnp.float32)],
    )(h1, w2, b2.reshape(1, _DM), res, g.reshape(1, _DM), be.reshape(1, _DM))


_NC, _NS = 2, 16
_NW = _NC * _NS
_BPW = _L // _NW


def _sc_gather_rows(table, idx):
    mesh = plsc.VectorSubcoreMesh(core_axis_name="c", subcore_axis_name="s")

    @functools.partial(
        pl.kernel, mesh=mesh,
        out_type=jax.ShapeDtypeStruct((_L, _DM), jnp.float32),
        scratch_types=[
            pltpu.VMEM((_BPW,), jnp.int32),
            pltpu.VMEM((_BPW, _DM), jnp.float32),
            pltpu.SemaphoreType.DMA,
        ],
    )
    def k(table_hbm, idx_hbm, out_hbm, idx_v, rows_v, sem):
        wid = jax.lax.axis_index("s") * _NC + jax.lax.axis_index("c")
        base = wid * _BPW
        pltpu.sync_copy(idx_hbm.at[pl.ds(base, _BPW)], idx_v)
        pltpu.async_copy(table_hbm.at[idx_v], rows_v, sem).wait()
        pltpu.sync_copy(rows_v, out_hbm.at[pl.ds(base, _BPW)])

    return k(table, idx)


def _embed_kern(g_ref, xe_ref, xm_ref, pe_ref, wv_ref, wm_ref, b_ref, o_ref):
    r = g_ref[...]
    r += xe_ref[...] * wv_ref[...]
    r += jnp.dot(xm_ref[...], wm_ref[...], preferred_element_type=jnp.float32)
    o_ref[...] = r + b_ref[...] + pe_ref[...]


def _embed(gathered, xe, xm, pe, wv, wm, bias):
    bm = 256
    grid = (_L // bm,)
    return pl.pallas_call(
        _embed_kern,
        grid=grid,
        in_specs=[
            pl.BlockSpec((bm, _DM), lambda i: (i, 0)),
            pl.BlockSpec((bm, 1), lambda i: (i, 0)),
            pl.BlockSpec((bm, 4), lambda i: (i, 0)),
            pl.BlockSpec((bm, _DM), lambda i: (i, 0)),
            pl.BlockSpec((1, _DM), lambda i: (0, 0)),
            pl.BlockSpec((4, _DM), lambda i: (0, 0)),
            pl.BlockSpec((1, _DM), lambda i: (0, 0)),
        ],
        out_specs=pl.BlockSpec((bm, _DM), lambda i: (i, 0)),
        out_shape=jax.ShapeDtypeStruct((_L, _DM), jnp.float32),
    )(gathered, xe, xm, pe, wv, wm, bias)


def _mscore_kern(k_ref, qt_ref, cnt_ref, o_ref, mx_ref, ms_ref, *, nj):
    j = pl.program_id(0)
    h = pl.program_id(1)
    kk = k_ref[0, 0]
    qt = qt_ref[0]
    s = jnp.dot(kk, qt, preferred_element_type=jnp.float32)
    c = cnt_ref[...]
    pm = jnp.max(jnp.where(c > 0.0, s, _NEG), axis=0, keepdims=True)
    ps = jnp.sum(s * c, axis=0, keepdims=True)

    @pl.when(j == 0)
    def _():
        mx_ref[pl.ds(h, 1), :] = pm
        ms_ref[pl.ds(h, 1), :] = ps

    @pl.when(j > 0)
    def _():
        mx_ref[pl.ds(h, 1), :] = jnp.maximum(mx_ref[pl.ds(h, 1), :], pm)
        ms_ref[pl.ds(h, 1), :] = ms_ref[pl.ds(h, 1), :] + ps

    @pl.when(j == nj - 1)
    def _():
        o_ref[pl.ds(h, 1), :] = (mx_ref[pl.ds(h, 1), :]
                                 - ms_ref[pl.ds(h, 1), :] * (1.0 / _L))


def _mscore(qkv, qt, cnt_t):
    bj = 256
    nj = _L // bj
    return pl.pallas_call(
        functools.partial(_mscore_kern, nj=nj),
        grid=(nj, _H),
        in_specs=[
            pl.BlockSpec((1, 1, bj, _DH), lambda j, h: (1, h, j, 0)),
            pl.BlockSpec((1, _DH, _L), lambda j, h: (h, 0, 0)),
            pl.BlockSpec((bj, _L), lambda j, h: (j, 0)),
        ],
        out_specs=pl.BlockSpec((_H, _L), lambda j, h: (0, 0)),
        out_shape=jax.ShapeDtypeStruct((_H, _L), jnp.float32),
        scratch_shapes=[pltpu.VMEM((_H, _L), jnp.float32),
                        pltpu.VMEM((_H, _L), jnp.float32)],
    )(qkv, qt, cnt_t)


def _topk_kern(m_ref, o_ref):
    m = m_ref[...]
    iota = jax.lax.broadcasted_iota(jnp.int32, (_H, _L), 1)
    for t in range(_U):
        cur = jnp.max(m, axis=1, keepdims=True)
        idx = jnp.min(jnp.where(m == cur, iota, _L), axis=1, keepdims=True)
        o_ref[:, t:t + 1] = idx
        m = jnp.where(iota == idx, _NEG, m)


def _topk(m):
    return pl.pallas_call(
        _topk_kern,
        grid=(1,),
        in_specs=[pl.BlockSpec((_H, _L), lambda i: (0, 0))],
        out_specs=pl.BlockSpec((_H, _U), lambda i: (0, 0)),
        out_shape=jax.ShapeDtypeStruct((_H, _U), jnp.int32),
    )(m)


_HB = 4


def _selattn_kern(q_ref, kt_ref, v_ref, tc_ref, tr_ref, o_ref):
    iota_c = jax.lax.broadcasted_iota(jnp.int32, (_U, _L), 1)
    iota_r = jax.lax.broadcasted_iota(jnp.int32, (_L, _U), 0)
    scs = []
    for hh in range(_HB):
        sel = (iota_c == tc_ref[0, hh]).astype(jnp.float32)
        qr = jnp.dot(sel, q_ref[0, hh], preferred_element_type=jnp.float32)
        scs.append(jnp.dot(qr, kt_ref[hh], preferred_element_type=jnp.float32))
    sc = jnp.concatenate(scs, axis=0) * (1.0 / 8.0)
    sc = sc - jnp.max(sc, axis=1, keepdims=True)
    e = jnp.exp(sc)
    attn = e / jnp.sum(e, axis=1, keepdims=True)
    for hh in range(_HB):
        a = attn[hh * _U:(hh + 1) * _U]
        v = v_ref[0, hh]
        upd = jnp.dot(a, v, preferred_element_type=jnp.float32)
        meanv = jnp.mean(v, axis=0, keepdims=True)
        sel_t = (iota_r == tr_ref[0, hh]).astype(jnp.float32)
        o_ref[hh] = meanv + jnp.dot(sel_t, upd - meanv,
                                    preferred_element_type=jnp.float32)


def _selattn(qkv, kt, tid_c, tid_r):
    return pl.pallas_call(
        _selattn_kern,
        grid=(_H // _HB,),
        in_specs=[
            pl.BlockSpec((1, _HB, _L, _DH), lambda h: (0, h, 0, 0)),
            pl.BlockSpec((_HB, _DH, _L), lambda h: (h, 0, 0)),
            pl.BlockSpec((1, _HB, _L, _DH), lambda h: (2, h, 0, 0)),
            pl.BlockSpec((1, _HB, _U, 1), lambda h: (0, h, 0, 0)),
            pl.BlockSpec((1, _HB, 1, _U), lambda h: (0, h, 0, 0)),
        ],
        out_specs=pl.BlockSpec((_HB, _L, _DH), lambda h: (h, 0, 0)),
        out_shape=jax.ShapeDtypeStruct((_H, _L, _DH), jnp.float32),
    )(qkv, kt, qkv, tid_c, tid_r)


def _addln_kern(x_ref, y_ref, g_ref, b_ref, o_ref):
    t = x_ref[...] + y_ref[...]
    mu = jnp.mean(t, axis=1, keepdims=True)
    d = t - mu
    var = jnp.mean(d * d, axis=1, keepdims=True)
    o_ref[...] = d * jax.lax.rsqrt(var + 1e-5) * g_ref[...] + b_ref[...]


def _addln(x, y, g, b):
    m = x.shape[0]
    bm = min(256, m)
    return pl.pallas_call(
        _addln_kern,
        grid=(m // bm,),
        in_specs=[
            pl.BlockSpec((bm, _DM), lambda i: (i, 0)),
            pl.BlockSpec((bm, _DM), lambda i: (i, 0)),
            pl.BlockSpec((1, _DM), lambda i: (0, 0)),
            pl.BlockSpec((1, _DM), lambda i: (0, 0)),
        ],
        out_specs=pl.BlockSpec((bm, _DM), lambda i: (i, 0)),
        out_shape=jax.ShapeDtypeStruct((m, _DM), jnp.float32),
    )(x, y, g.reshape(1, _DM), b.reshape(1, _DM))


def _final_kern(x_ref, g_ref, b_ref, wt_ref, o_ref):
    t = x_ref[...]
    mu = jnp.mean(t, axis=1, keepdims=True)
    d = t - mu
    var = jnp.mean(d * d, axis=1, keepdims=True)
    n = d * jax.lax.rsqrt(var + 1e-5) * g_ref[...] + b_ref[...]
    o_ref[...] = jnp.dot(n, wt_ref[...], preferred_element_type=jnp.float32)


def _final(x8, g, b, emb_t):
    return pl.pallas_call(
        _final_kern,
        grid=(1,),
        in_specs=[
            pl.BlockSpec((8, _DM), lambda i: (0, 0)),
            pl.BlockSpec((1, _DM), lambda i: (0, 0)),
            pl.BlockSpec((1, _DM), lambda i: (0, 0)),
            pl.BlockSpec((_DM, _NSTATE), lambda i: (0, 0)),
        ],
        out_specs=pl.BlockSpec((8, _NSTATE), lambda i: (0, 0)),
        out_shape=jax.ShapeDtypeStruct((8, _NSTATE), jnp.float32),
    )(x8, g.reshape(1, _DM), b.reshape(1, _DM), emb_t)


def _attention_ctx(x, p, cnt_t):
    wqkv = jnp.concatenate([p['Wq'], p['Wk'], p['Wv']], axis=1)
    bqkv = jnp.concatenate([p['bq'], p['bk'], p['bv']])
    qkv = _qkv(x, wqkv, bqkv)
    qt = jnp.swapaxes(qkv[0], 1, 2)
    kt = jnp.swapaxes(qkv[1], 1, 2)
    m = _mscore(qkv, qt, cnt_t)
    tid = _topk(m)
    ctx = _selattn(qkv, kt, tid.reshape(1, _H, _U, 1),
                   tid.reshape(1, _H, 1, _U))
    return ctx


def _layer_full(x, p, cnt_t):
    ctx = _attention_ctx(x, p, cnt_t)
    x1 = _attnout_ln(ctx, p['Wo'], p['bo'], x, p['g1'], p['be1'])
    h1 = _ffn1(x1, p['W1'], p['b1'])
    return _ffn2_ln(h1, p['W2'], p['b2'], x1, p['g2'], p['be2'])


def _layer_last8(x, p, cnt_t):
    ctx = _attention_ctx(x, p, cnt_t)
    ctx8 = ctx[:, _L - 8:, :].transpose(1, 0, 2).reshape(8, _DM)
    x8 = x[_L - 8:]
    attn8 = _matmul(ctx8, p['Wo'], p['bo'], bm=8)
    x1 = _addln(x8, attn8, p['g1'], p['be1'])
    h1 = _matmul(x1, p['W1'], p['b1'], act="gelu", bm=8)
    h2 = _matmul(h1, p['W2'], p['b2'], bm=8)
    return _addln(x1, h2, p['g2'], p['be2'])


def kernel(x_state_enc, x_enc, x_mark_enc, params):
    p = params
    st = x_state_enc.reshape(_L, 1).astype(jnp.int32)
    xe = x_enc.reshape(_L, 1).astype(jnp.float32)
    xm = x_mark_enc.reshape(_L, 4).astype(jnp.float32)

    bias0 = (p['b_val'] + p['b_mark']).reshape(1, _DM)
    pe = jnp.asarray(_PE)

    gathered = _sc_gather_rows(p['state_emb'], st.reshape(_L))
    x = _embed(gathered, xe, xm, pe, p['W_val'], p['W_mark'], bias0)
    x = _layer_full(x, p['layers'][0], jnp.asarray(_CNT_T[0]))
    x8 = _layer_last8(x, p['layers'][1], jnp.asarray(_CNT_T[1]))
    y = _final(x8, p['norm_g'], p['norm_b'], p['state_emb'].T)
    return y[7:8, :_NSTATE - 1].reshape(1, 1, _NSTATE - 1)


# speedup vs baseline: 1.1187x; 1.1187x over previous
import functools

import numpy as np
import jax
import jax.numpy as jnp
from jax.experimental import pallas as pl
from jax.experimental.pallas import tpu as pltpu
from jax.experimental.pallas import tpu_sc as plsc

_L = 2048
_DM = 1024
_H = 16
_DH = 64
_DFF = 4096
_NSTATE = 65
_FACTOR = 5
_U = min(int(_FACTOR * np.ceil(np.log(_L))), _L)
_NEG = -1e30


def _pe_table():
    pe = np.zeros((_L, _DM), np.float32)
    pos = np.arange(_L, dtype=np.float32)[:, None]
    div = np.exp(np.arange(0, _DM, 2, dtype=np.float32) * (-np.log(10000.0) / _DM))
    pe[:, 0::2] = np.sin(pos * div)
    pe[:, 1::2] = np.cos(pos * div)
    return pe


_PE = _pe_table()


def _sample_count_matrix_t(layer_idx):
    rng = jax.random.fold_in(jax.random.key(42), layer_idx)
    idx = np.asarray(jax.random.randint(rng, (_L, _U), 0, _L))
    cnt_t = np.zeros((_L, _L), np.float32)
    np.add.at(cnt_t, (idx, np.arange(_L)[:, None]), 1.0)
    return cnt_t


_CNT_T = [_sample_count_matrix_t(li) for li in range(2)]


def _mm_kern(x_ref, w_ref, b_ref, o_ref, acc_ref, *, nk, act):
    @pl.when(pl.program_id(2) == 0)
    def _():
        acc_ref[...] = jnp.zeros_like(acc_ref)

    acc_ref[...] += jnp.dot(x_ref[...], w_ref[...],
                            preferred_element_type=jnp.float32)

    @pl.when(pl.program_id(2) == nk - 1)
    def _():
        r = acc_ref[...] + b_ref[...]
        if act == "gelu":
            r = jax.nn.gelu(r)
        o_ref[...] = r


def _matmul(x, w, b, act=None, bm=256, bn=512, bk=512):
    m, k = x.shape
    n = w.shape[1]
    bm, bn, bk = min(bm, m), min(bn, n), min(bk, k)
    grid = (m // bm, n // bn, k // bk)
    return pl.pallas_call(
        functools.partial(_mm_kern, nk=grid[2], act=act),
        grid=grid,
        in_specs=[
            pl.BlockSpec((bm, bk), lambda i, j, kk: (i, kk)),
            pl.BlockSpec((bk, bn), lambda i, j, kk: (kk, j)),
            pl.BlockSpec((1, bn), lambda i, j, kk: (0, j)),
        ],
        out_specs=pl.BlockSpec((bm, bn), lambda i, j, kk: (i, j)),
        out_shape=jax.ShapeDtypeStruct((m, n), jnp.float32),
        scratch_shapes=[pltpu.VMEM((bm, bn), jnp.float32)],
    )(x, w, b.reshape(1, n))


def _qkv_kern(x_ref, w_ref, b_ref, o_ref):
    acc = jnp.dot(x_ref[...], w_ref[...],
                  preferred_element_type=jnp.float32) + b_ref[...]
    o_ref[0, 0] = acc[:, :_DH]
    o_ref[0, 1] = acc[:, _DH:]


def _qkv(x, wqkv, bqkv):
    bm = 1024
    grid = (_L // bm, 3, _H // 2)
    return pl.pallas_call(
        _qkv_kern,
        grid=grid,
        in_specs=[
            pl.BlockSpec((bm, _DM), lambda i, t, j: (i, 0)),
            pl.BlockSpec((_DM, 2 * _DH), lambda i, t, j: (0, t * (_H // 2) + j)),
            pl.BlockSpec((1, 2 * _DH), lambda i, t, j: (0, t * (_H // 2) + j)),
        ],
        out_specs=pl.BlockSpec((1, 2, bm, _DH), lambda i, t, j: (t, j, i, 0)),
        out_shape=jax.ShapeDtypeStruct((3, _H, _L, _DH), jnp.float32),
    )(x, wqkv, bqkv.reshape(1, 3 * _DM))


def _attnout_ln_kern(ctx_ref, w_ref, b_ref, res_ref, g_ref, be_ref, o_ref,
                     acc_ref, *, nk):
    kk = pl.program_id(1)

    @pl.when(kk == 0)
    def _():
        acc_ref[...] = jnp.zeros_like(acc_ref)

    c = ctx_ref[...]
    xcat = jnp.concatenate([c[0], c[1]], axis=1)
    acc_ref[...] += jnp.dot(xcat, w_ref[...], preferred_element_type=jnp.float32)

    @pl.when(kk == nk - 1)
    def _():
        t = acc_ref[...] + b_ref[...] + res_ref[...]
        mu = jnp.mean(t, axis=1, keepdims=True)
        d = t - mu
        var = jnp.mean(d * d, axis=1, keepdims=True)
        o_ref[...] = d * jax.lax.rsqrt(var + 1e-5) * g_ref[...] + be_ref[...]


def _attnout_ln(ctx, wo, bo, res, g, be):
    bm = 256
    nk = _H // 2
    return pl.pallas_call(
        functools.partial(_attnout_ln_kern, nk=nk),
        grid=(_L // bm, nk),
        in_specs=[
            pl.BlockSpec((2, bm, _DH), lambda i, kk: (kk, i, 0)),
            pl.BlockSpec((2 * _DH, _DM), lambda i, kk: (kk, 0)),
            pl.BlockSpec((1, _DM), lambda i, kk: (0, 0)),
            pl.BlockSpec((bm, _DM), lambda i, kk: (i, 0)),
            pl.BlockSpec((1, _DM), lambda i, kk: (0, 0)),
            pl.BlockSpec((1, _DM), lambda i, kk: (0, 0)),
        ],
        out_specs=pl.BlockSpec((bm, _DM), lambda i, kk: (i, 0)),
        out_shape=jax.ShapeDtypeStruct((_L, _DM), jnp.float32),
        scratch_shapes=[pltpu.VMEM((bm, _DM), jnp.float32)],
    )(ctx, wo, bo.reshape(1, _DM), res, g.reshape(1, _DM), be.reshape(1, _DM))


def _ffn1_kern(x_ref, w_ref, b_ref, o_ref):
    o_ref[...] = jax.nn.gelu(
        jnp.dot(x_ref[...], w_ref[...], preferred_element_type=jnp.float32)
        + b_ref[...])


def _ffn1(x, w1, b1):
    bm, bn = 1024, 512
    return pl.pallas_call(
        _ffn1_kern,
        grid=(_L // bm, _DFF // bn),
        in_specs=[
            pl.BlockSpec((bm, _DM), lambda i, j: (i, 0)),
            pl.BlockSpec((_DM, bn), lambda i, j: (0, j)),
            pl.BlockSpec((1, bn), lambda i, j: (0, j)),
        ],
        out_specs=pl.BlockSpec((bm, bn), lambda i, j: (i, j)),
        out_shape=jax.ShapeDtypeStruct((_L, _DFF), jnp.float32),
    )(x, w1, b1.reshape(1, _DFF))


def _ffn2_ln_kern(x_ref, w_ref, b_ref, res_ref, g_ref, be_ref, o_ref,
                  acc_ref, *, nk):
    kk = pl.program_id(1)

    @pl.when(kk == 0)
    def _():
        acc_ref[...] = jnp.zeros_like(acc_ref)

    acc_ref[...] += jnp.dot(x_ref[...], w_ref[...],
                            preferred_element_type=jnp.float32)

    @pl.when(kk == nk - 1)
    def _():
        t = acc_ref[...] + b_ref[...] + res_ref[...]
        mu = jnp.mean(t, axis=1, keepdims=True)
        d = t - mu
        var = jnp.mean(d * d, axis=1, keepdims=True)
        o_ref[...] = d * jax.lax.rsqrt(var + 1e-5) * g_ref[...] + be_ref[...]


def _ffn2_ln(h1, w2, b2, res, g, be):
    bm, bk = 512, 512
    nk = _DFF // bk
    return pl.pallas_call(
        functools.partial(_ffn2_ln_kern, nk=nk),
        grid=(_L // bm, nk),
        in_specs=[
            pl.BlockSpec((bm, bk), lambda i, kk: (i, kk)),
            pl.BlockSpec((bk, _DM), lambda i, kk: (kk, 0)),
            pl.BlockSpec((1, _DM), lambda i, kk: (0, 0)),
            pl.BlockSpec((bm, _DM), lambda i, kk: (i, 0)),
            pl.BlockSpec((1, _DM), lambda i, kk: (0, 0)),
            pl.BlockSpec((1, _DM), lambda i, kk: (0, 0)),
        ],
        out_specs=pl.BlockSpec((bm, _DM), lambda i, kk: (i, 0)),
        out_shape=jax.ShapeDtypeStruct((_L, _DM), jnp.float32),
        scratch_shapes=[pltpu.VMEM((bm, _DM), jnp.float32)],
    )(h1, w2, b2.reshape(1, _DM), res, g.reshape(1, _DM), be.reshape(1, _DM))


_NC, _NS = 2, 16
_NW = _NC * _NS
_BPW = _L // _NW


def _sc_gather_rows(table, idx):
    mesh = plsc.VectorSubcoreMesh(core_axis_name="c", subcore_axis_name="s")

    @functools.partial(
        pl.kernel, mesh=mesh,
        out_type=jax.ShapeDtypeStruct((_L, _DM), jnp.float32),
        scratch_types=[
            pltpu.VMEM((_BPW,), jnp.int32),
            pltpu.VMEM((_BPW, _DM), jnp.float32),
            pltpu.SemaphoreType.DMA,
        ],
    )
    def k(table_hbm, idx_hbm, out_hbm, idx_v, rows_v, sem):
        wid = jax.lax.axis_index("s") * _NC + jax.lax.axis_index("c")
        base = wid * _BPW
        pltpu.sync_copy(idx_hbm.at[pl.ds(base, _BPW)], idx_v)
        pltpu.async_copy(table_hbm.at[idx_v], rows_v, sem).wait()
        pltpu.sync_copy(rows_v, out_hbm.at[pl.ds(base, _BPW)])

    return k(table, idx)


def _embed_kern(g_ref, xe_ref, xm_ref, pe_ref, wv_ref, wm_ref, b_ref, o_ref):
    r = g_ref[...]
    r += xe_ref[...] * wv_ref[...]
    r += jnp.dot(xm_ref[...], wm_ref[...], preferred_element_type=jnp.float32)
    o_ref[...] = r + b_ref[...] + pe_ref[...]


def _embed(gathered, xe, xm, pe, wv, wm, bias):
    bm = 256
    grid = (_L // bm,)
    return pl.pallas_call(
        _embed_kern,
        grid=grid,
        in_specs=[
            pl.BlockSpec((bm, _DM), lambda i: (i, 0)),
            pl.BlockSpec((bm, 1), lambda i: (i, 0)),
            pl.BlockSpec((bm, 4), lambda i: (i, 0)),
            pl.BlockSpec((bm, _DM), lambda i: (i, 0)),
            pl.BlockSpec((1, _DM), lambda i: (0, 0)),
            pl.BlockSpec((4, _DM), lambda i: (0, 0)),
            pl.BlockSpec((1, _DM), lambda i: (0, 0)),
        ],
        out_specs=pl.BlockSpec((bm, _DM), lambda i: (i, 0)),
        out_shape=jax.ShapeDtypeStruct((_L, _DM), jnp.float32),
    )(gathered, xe, xm, pe, wv, wm, bias)


def _mscore_kern(k_ref, qt_ref, cnt_ref, o_ref, mx_ref, ms_ref, *, nj):
    j = pl.program_id(0)
    h = pl.program_id(1)
    kk = k_ref[0, 0]
    qt = qt_ref[0]
    s = jnp.dot(kk, qt, preferred_element_type=jnp.float32)
    c = cnt_ref[...]
    pm = jnp.max(jnp.where(c > 0.0, s, _NEG), axis=0, keepdims=True)
    ps = jnp.sum(s * c, axis=0, keepdims=True)

    @pl.when(j == 0)
    def _():
        mx_ref[pl.ds(h, 1), :] = pm
        ms_ref[pl.ds(h, 1), :] = ps

    @pl.when(j > 0)
    def _():
        mx_ref[pl.ds(h, 1), :] = jnp.maximum(mx_ref[pl.ds(h, 1), :], pm)
        ms_ref[pl.ds(h, 1), :] = ms_ref[pl.ds(h, 1), :] + ps

    @pl.when(j == nj - 1)
    def _():
        o_ref[pl.ds(h, 1), :] = (mx_ref[pl.ds(h, 1), :]
                                 - ms_ref[pl.ds(h, 1), :] * (1.0 / _L))


def _mscore(qkv, qt, cnt_t):
    bj = 512
    nj = _L // bj
    return pl.pallas_call(
        functools.partial(_mscore_kern, nj=nj),
        grid=(nj, _H),
        in_specs=[
            pl.BlockSpec((1, 1, bj, _DH), lambda j, h: (1, h, j, 0)),
            pl.BlockSpec((1, _DH, _L), lambda j, h: (h, 0, 0)),
            pl.BlockSpec((bj, _L), lambda j, h: (j, 0)),
        ],
        out_specs=pl.BlockSpec((_H, _L), lambda j, h: (0, 0)),
        out_shape=jax.ShapeDtypeStruct((_H, _L), jnp.float32),
        scratch_shapes=[pltpu.VMEM((_H, _L), jnp.float32),
                        pltpu.VMEM((_H, _L), jnp.float32)],
    )(qkv, qt, cnt_t)


def _topk_kern(m_ref, o_ref):
    m = m_ref[...]
    iota = jax.lax.broadcasted_iota(jnp.int32, (_H, _L), 1)
    for t in range(_U):
        cur = jnp.max(m, axis=1, keepdims=True)
        idx = jnp.min(jnp.where(m == cur, iota, _L), axis=1, keepdims=True)
        o_ref[:, t:t + 1] = idx
        m = jnp.where(iota == idx, _NEG, m)


def _topk(m):
    return pl.pallas_call(
        _topk_kern,
        grid=(1,),
        in_specs=[pl.BlockSpec((_H, _L), lambda i: (0, 0))],
        out_specs=pl.BlockSpec((_H, _U), lambda i: (0, 0)),
        out_shape=jax.ShapeDtypeStruct((_H, _U), jnp.int32),
    )(m)


_HB = 4


def _selattn_kern(q_ref, kt_ref, v_ref, tc_ref, tr_ref, o_ref):
    iota_c = jax.lax.broadcasted_iota(jnp.int32, (_U, _L), 1)
    iota_r = jax.lax.broadcasted_iota(jnp.int32, (_L, _U), 0)
    scs = []
    for hh in range(_HB):
        sel = (iota_c == tc_ref[0, hh]).astype(jnp.float32)
        qr = jnp.dot(sel, q_ref[0, hh], preferred_element_type=jnp.float32)
        scs.append(jnp.dot(qr, kt_ref[hh], preferred_element_type=jnp.float32))
    sc = jnp.concatenate(scs, axis=0) * (1.0 / 8.0)
    sc = sc - jnp.max(sc, axis=1, keepdims=True)
    e = jnp.exp(sc)
    attn = e / jnp.sum(e, axis=1, keepdims=True)
    for hh in range(_HB):
        a = attn[hh * _U:(hh + 1) * _U]
        v = v_ref[0, hh]
        upd = jnp.dot(a, v, preferred_element_type=jnp.float32)
        meanv = jnp.mean(v, axis=0, keepdims=True)
        sel_t = (iota_r == tr_ref[0, hh]).astype(jnp.float32)
        o_ref[hh] = meanv + jnp.dot(sel_t, upd - meanv,
                                    preferred_element_type=jnp.float32)


def _selattn(qkv, kt, tid_c, tid_r):
    return pl.pallas_call(
        _selattn_kern,
        grid=(_H // _HB,),
        in_specs=[
            pl.BlockSpec((1, _HB, _L, _DH), lambda h: (0, h, 0, 0)),
            pl.BlockSpec((_HB, _DH, _L), lambda h: (h, 0, 0)),
            pl.BlockSpec((1, _HB, _L, _DH), lambda h: (2, h, 0, 0)),
            pl.BlockSpec((1, _HB, _U, 1), lambda h: (0, h, 0, 0)),
            pl.BlockSpec((1, _HB, 1, _U), lambda h: (0, h, 0, 0)),
        ],
        out_specs=pl.BlockSpec((_HB, _L, _DH), lambda h: (h, 0, 0)),
        out_shape=jax.ShapeDtypeStruct((_H, _L, _DH), jnp.float32),
    )(qkv, kt, qkv, tid_c, tid_r)


def _addln_kern(x_ref, y_ref, g_ref, b_ref, o_ref):
    t = x_ref[...] + y_ref[...]
    mu = jnp.mean(t, axis=1, keepdims=True)
    d = t - mu
    var = jnp.mean(d * d, axis=1, keepdims=True)
    o_ref[...] = d * jax.lax.rsqrt(var + 1e-5) * g_ref[...] + b_ref[...]


def _addln(x, y, g, b):
    m = x.shape[0]
    bm = min(256, m)
    return pl.pallas_call(
        _addln_kern,
        grid=(m // bm,),
        in_specs=[
            pl.BlockSpec((bm, _DM), lambda i: (i, 0)),
            pl.BlockSpec((bm, _DM), lambda i: (i, 0)),
            pl.BlockSpec((1, _DM), lambda i: (0, 0)),
            pl.BlockSpec((1, _DM), lambda i: (0, 0)),
        ],
        out_specs=pl.BlockSpec((bm, _DM), lambda i: (i, 0)),
        out_shape=jax.ShapeDtypeStruct((m, _DM), jnp.float32),
    )(x, y, g.reshape(1, _DM), b.reshape(1, _DM))


def _final_kern(x_ref, g_ref, b_ref, wt_ref, o_ref):
    t = x_ref[...]
    mu = jnp.mean(t, axis=1, keepdims=True)
    d = t - mu
    var = jnp.mean(d * d, axis=1, keepdims=True)
    n = d * jax.lax.rsqrt(var + 1e-5) * g_ref[...] + b_ref[...]
    o_ref[...] = jnp.dot(n, wt_ref[...], preferred_element_type=jnp.float32)


def _final(x8, g, b, emb_t):
    return pl.pallas_call(
        _final_kern,
        grid=(1,),
        in_specs=[
            pl.BlockSpec((8, _DM), lambda i: (0, 0)),
            pl.BlockSpec((1, _DM), lambda i: (0, 0)),
            pl.BlockSpec((1, _DM), lambda i: (0, 0)),
            pl.BlockSpec((_DM, _NSTATE), lambda i: (0, 0)),
        ],
        out_specs=pl.BlockSpec((8, _NSTATE), lambda i: (0, 0)),
        out_shape=jax.ShapeDtypeStruct((8, _NSTATE), jnp.float32),
    )(x8, g.reshape(1, _DM), b.reshape(1, _DM), emb_t)


def _attention_ctx(x, p, cnt_t):
    wqkv = jnp.concatenate([p['Wq'], p['Wk'], p['Wv']], axis=1)
    bqkv = jnp.concatenate([p['bq'], p['bk'], p['bv']])
    qkv = _qkv(x, wqkv, bqkv)
    qt = jnp.swapaxes(qkv[0], 1, 2)
    kt = jnp.swapaxes(qkv[1], 1, 2)
    m = _mscore(qkv, qt, cnt_t)
    tid = _topk(m)
    ctx = _selattn(qkv, kt, tid.reshape(1, _H, _U, 1),
                   tid.reshape(1, _H, 1, _U))
    return ctx


def _layer_full(x, p, cnt_t):
    ctx = _attention_ctx(x, p, cnt_t)
    x1 = _attnout_ln(ctx, p['Wo'], p['bo'], x, p['g1'], p['be1'])
    h1 = _ffn1(x1, p['W1'], p['b1'])
    return _ffn2_ln(h1, p['W2'], p['b2'], x1, p['g2'], p['be2'])


def _layer_last8(x, p, cnt_t):
    ctx = _attention_ctx(x, p, cnt_t)
    ctx8 = ctx[:, _L - 8:, :].transpose(1, 0, 2).reshape(8, _DM)
    x8 = x[_L - 8:]
    attn8 = _matmul(ctx8, p['Wo'], p['bo'], bm=8)
    x1 = _addln(x8, attn8, p['g1'], p['be1'])
    h1 = _matmul(x1, p['W1'], p['b1'], act="gelu", bm=8)
    h2 = _matmul(h1, p['W2'], p['b2'], bm=8)
    return _addln(x1, h2, p['g2'], p['be2'])


def kernel(x_state_enc, x_enc, x_mark_enc, params):
    p = params
    st = x_state_enc.reshape(_L, 1).astype(jnp.int32)
    xe = x_enc.reshape(_L, 1).astype(jnp.float32)
    xm = x_mark_enc.reshape(_L, 4).astype(jnp.float32)

    bias0 = (p['b_val'] + p['b_mark']).reshape(1, _DM)
    pe = jnp.asarray(_PE)

    gathered = _sc_gather_rows(p['state_emb'], st.reshape(_L))
    x = _embed(gathered, xe, xm, pe, p['W_val'], p['W_mark'], bias0)
    x = _layer_full(x, p['layers'][0], jnp.asarray(_CNT_T[0]))
    x8 = _layer_last8(x, p['layers'][1], jnp.asarray(_CNT_T[1]))
    y = _final(x8, p['norm_g'], p['norm_b'], p['state_emb'].T)
    return y[7:8, :_NSTATE - 1].reshape(1, 1, _NSTATE - 1)


# speedup vs baseline: 1.1694x; 1.0453x over previous
import functools

import numpy as np
import jax
import jax.numpy as jnp
from jax.experimental import pallas as pl
from jax.experimental.pallas import tpu as pltpu
from jax.experimental.pallas import tpu_sc as plsc

_L = 2048
_DM = 1024
_H = 16
_DH = 64
_DFF = 4096
_NSTATE = 65
_FACTOR = 5
_U = min(int(_FACTOR * np.ceil(np.log(_L))), _L)
_NEG = -1e30


def _pe_table():
    pe = np.zeros((_L, _DM), np.float32)
    pos = np.arange(_L, dtype=np.float32)[:, None]
    div = np.exp(np.arange(0, _DM, 2, dtype=np.float32) * (-np.log(10000.0) / _DM))
    pe[:, 0::2] = np.sin(pos * div)
    pe[:, 1::2] = np.cos(pos * div)
    return pe


_PE = _pe_table()


def _sample_count_matrix_t(layer_idx):
    rng = jax.random.fold_in(jax.random.key(42), layer_idx)
    idx = np.asarray(jax.random.randint(rng, (_L, _U), 0, _L))
    cnt_t = np.zeros((_L, _L), np.float32)
    np.add.at(cnt_t, (idx, np.arange(_L)[:, None]), 1.0)
    return cnt_t


_CNT_T = [_sample_count_matrix_t(li) for li in range(2)]


def _mm_kern(x_ref, w_ref, b_ref, o_ref, acc_ref, *, nk, act):
    @pl.when(pl.program_id(2) == 0)
    def _():
        acc_ref[...] = jnp.zeros_like(acc_ref)

    acc_ref[...] += jnp.dot(x_ref[...], w_ref[...],
                            preferred_element_type=jnp.float32)

    @pl.when(pl.program_id(2) == nk - 1)
    def _():
        r = acc_ref[...] + b_ref[...]
        if act == "gelu":
            r = jax.nn.gelu(r)
        o_ref[...] = r


def _matmul(x, w, b, act=None, bm=256, bn=512, bk=512):
    m, k = x.shape
    n = w.shape[1]
    bm, bn, bk = min(bm, m), min(bn, n), min(bk, k)
    grid = (m // bm, n // bn, k // bk)
    return pl.pallas_call(
        functools.partial(_mm_kern, nk=grid[2], act=act),
        grid=grid,
        in_specs=[
            pl.BlockSpec((bm, bk), lambda i, j, kk: (i, kk)),
            pl.BlockSpec((bk, bn), lambda i, j, kk: (kk, j)),
            pl.BlockSpec((1, bn), lambda i, j, kk: (0, j)),
        ],
        out_specs=pl.BlockSpec((bm, bn), lambda i, j, kk: (i, j)),
        out_shape=jax.ShapeDtypeStruct((m, n), jnp.float32),
        scratch_shapes=[pltpu.VMEM((bm, bn), jnp.float32)],
    )(x, w, b.reshape(1, n))


def _qkv_kern(x_ref, w_ref, b_ref, o_ref):
    acc = jnp.dot(x_ref[...], w_ref[...],
                  preferred_element_type=jnp.float32) + b_ref[...]
    o_ref[0, 0] = acc[:, :_DH]
    o_ref[0, 1] = acc[:, _DH:]


def _qkv(x, wqkv, bqkv):
    bm = 1024
    grid = (_L // bm, 3, _H // 2)
    return pl.pallas_call(
        _qkv_kern,
        grid=grid,
        in_specs=[
            pl.BlockSpec((bm, _DM), lambda i, t, j: (i, 0)),
            pl.BlockSpec((_DM, 2 * _DH), lambda i, t, j: (0, t * (_H // 2) + j)),
            pl.BlockSpec((1, 2 * _DH), lambda i, t, j: (0, t * (_H // 2) + j)),
        ],
        out_specs=pl.BlockSpec((1, 2, bm, _DH), lambda i, t, j: (t, j, i, 0)),
        out_shape=jax.ShapeDtypeStruct((3, _H, _L, _DH), jnp.float32),
    )(x, wqkv, bqkv.reshape(1, 3 * _DM))


def _attnout_ln_kern(ctx_ref, w_ref, b_ref, res_ref, g_ref, be_ref, o_ref,
                     acc_ref, *, nk):
    kk = pl.program_id(1)

    @pl.when(kk == 0)
    def _():
        acc_ref[...] = jnp.zeros_like(acc_ref)

    c = ctx_ref[...]
    xcat = jnp.concatenate([c[0], c[1]], axis=1)
    acc_ref[...] += jnp.dot(xcat, w_ref[...], preferred_element_type=jnp.float32)

    @pl.when(kk == nk - 1)
    def _():
        t = acc_ref[...] + b_ref[...] + res_ref[...]
        mu = jnp.mean(t, axis=1, keepdims=True)
        d = t - mu
        var = jnp.mean(d * d, axis=1, keepdims=True)
        o_ref[...] = d * jax.lax.rsqrt(var + 1e-5) * g_ref[...] + be_ref[...]


def _attnout_ln(ctx, wo, bo, res, g, be):
    bm = 256
    nk = _H // 2
    return pl.pallas_call(
        functools.partial(_attnout_ln_kern, nk=nk),
        grid=(_L // bm, nk),
        in_specs=[
            pl.BlockSpec((2, bm, _DH), lambda i, kk: (kk, i, 0)),
            pl.BlockSpec((2 * _DH, _DM), lambda i, kk: (kk, 0)),
            pl.BlockSpec((1, _DM), lambda i, kk: (0, 0)),
            pl.BlockSpec((bm, _DM), lambda i, kk: (i, 0)),
            pl.BlockSpec((1, _DM), lambda i, kk: (0, 0)),
            pl.BlockSpec((1, _DM), lambda i, kk: (0, 0)),
        ],
        out_specs=pl.BlockSpec((bm, _DM), lambda i, kk: (i, 0)),
        out_shape=jax.ShapeDtypeStruct((_L, _DM), jnp.float32),
        scratch_shapes=[pltpu.VMEM((bm, _DM), jnp.float32)],
    )(ctx, wo, bo.reshape(1, _DM), res, g.reshape(1, _DM), be.reshape(1, _DM))


def _ffn1_kern(x_ref, w_ref, b_ref, o_ref):
    o_ref[...] = jax.nn.gelu(
        jnp.dot(x_ref[...], w_ref[...], preferred_element_type=jnp.float32)
        + b_ref[...])


def _ffn1(x, w1, b1):
    bm, bn = 1024, 512
    return pl.pallas_call(
        _ffn1_kern,
        grid=(_L // bm, _DFF // bn),
        in_specs=[
            pl.BlockSpec((bm, _DM), lambda i, j: (i, 0)),
            pl.BlockSpec((_DM, bn), lambda i, j: (0, j)),
            pl.BlockSpec((1, bn), lambda i, j: (0, j)),
        ],
        out_specs=pl.BlockSpec((bm, bn), lambda i, j: (i, j)),
        out_shape=jax.ShapeDtypeStruct((_L, _DFF), jnp.float32),
    )(x, w1, b1.reshape(1, _DFF))


def _ffn2_ln_kern(x_ref, w_ref, b_ref, res_ref, g_ref, be_ref, o_ref,
                  acc_ref, *, nk):
    kk = pl.program_id(1)

    @pl.when(kk == 0)
    def _():
        acc_ref[...] = jnp.zeros_like(acc_ref)

    acc_ref[...] += jnp.dot(x_ref[...], w_ref[...],
                            preferred_element_type=jnp.float32)

    @pl.when(kk == nk - 1)
    def _():
        t = acc_ref[...] + b_ref[...] + res_ref[...]
        mu = jnp.mean(t, axis=1, keepdims=True)
        d = t - mu
        var = jnp.mean(d * d, axis=1, keepdims=True)
        o_ref[...] = d * jax.lax.rsqrt(var + 1e-5) * g_ref[...] + be_ref[...]


def _ffn2_ln(h1, w2, b2, res, g, be):
    bm, bk = 512, 512
    nk = _DFF // bk
    return pl.pallas_call(
        functools.partial(_ffn2_ln_kern, nk=nk),
        grid=(_L // bm, nk),
        in_specs=[
            pl.BlockSpec((bm, bk), lambda i, kk: (i, kk)),
            pl.BlockSpec((bk, _DM), lambda i, kk: (kk, 0)),
            pl.BlockSpec((1, _DM), lambda i, kk: (0, 0)),
            pl.BlockSpec((bm, _DM), lambda i, kk: (i, 0)),
            pl.BlockSpec((1, _DM), lambda i, kk: (0, 0)),
            pl.BlockSpec((1, _DM), lambda i, kk: (0, 0)),
        ],
        out_specs=pl.BlockSpec((bm, _DM), lambda i, kk: (i, 0)),
        out_shape=jax.ShapeDtypeStruct((_L, _DM), jnp.float32),
        scratch_shapes=[pltpu.VMEM((bm, _DM), jnp.float32)],
    )(h1, w2, b2.reshape(1, _DM), res, g.reshape(1, _DM), be.reshape(1, _DM))


_NC, _NS = 2, 16
_NW = _NC * _NS
_BPW = _L // _NW


def _sc_gather_rows(table, idx):
    mesh = plsc.VectorSubcoreMesh(core_axis_name="c", subcore_axis_name="s")

    @functools.partial(
        pl.kernel, mesh=mesh,
        out_type=jax.ShapeDtypeStruct((_L, _DM), jnp.float32),
        scratch_types=[
            pltpu.VMEM((_BPW,), jnp.int32),
            pltpu.VMEM((_BPW, _DM), jnp.float32),
            pltpu.SemaphoreType.DMA,
        ],
    )
    def k(table_hbm, idx_hbm, out_hbm, idx_v, rows_v, sem):
        wid = jax.lax.axis_index("s") * _NC + jax.lax.axis_index("c")
        base = wid * _BPW
        pltpu.sync_copy(idx_hbm.at[pl.ds(base, _BPW)], idx_v)
        pltpu.async_copy(table_hbm.at[idx_v], rows_v, sem).wait()
        pltpu.sync_copy(rows_v, out_hbm.at[pl.ds(base, _BPW)])

    return k(table, idx)


def _embed_kern(g_ref, xe_ref, xm_ref, pe_ref, wv_ref, wm_ref, b_ref, o_ref):
    r = g_ref[...]
    r += xe_ref[...] * wv_ref[...]
    r += jnp.dot(xm_ref[...], wm_ref[...], preferred_element_type=jnp.float32)
    o_ref[...] = r + b_ref[...] + pe_ref[...]


def _embed(gathered, xe, xm, pe, wv, wm, bias):
    bm = 256
    grid = (_L // bm,)
    return pl.pallas_call(
        _embed_kern,
        grid=grid,
        in_specs=[
            pl.BlockSpec((bm, _DM), lambda i: (i, 0)),
            pl.BlockSpec((bm, 1), lambda i: (i, 0)),
            pl.BlockSpec((bm, 4), lambda i: (i, 0)),
            pl.BlockSpec((bm, _DM), lambda i: (i, 0)),
            pl.BlockSpec((1, _DM), lambda i: (0, 0)),
            pl.BlockSpec((4, _DM), lambda i: (0, 0)),
            pl.BlockSpec((1, _DM), lambda i: (0, 0)),
        ],
        out_specs=pl.BlockSpec((bm, _DM), lambda i: (i, 0)),
        out_shape=jax.ShapeDtypeStruct((_L, _DM), jnp.float32),
    )(gathered, xe, xm, pe, wv, wm, bias)


def _mscore_kern(k_ref, qt_ref, cnt_ref, o_ref, mx_ref, ms_ref, *, nj):
    j = pl.program_id(0)
    h = pl.program_id(1)
    kk = k_ref[0, 0]
    qt = qt_ref[0]
    s = jnp.dot(kk, qt, preferred_element_type=jnp.float32)
    c = cnt_ref[...]
    pm = jnp.max(jnp.where(c > 0.0, s, _NEG), axis=0, keepdims=True)
    ps = jnp.sum(s * c, axis=0, keepdims=True)

    @pl.when(j == 0)
    def _():
        mx_ref[pl.ds(h, 1), :] = pm
        ms_ref[pl.ds(h, 1), :] = ps

    @pl.when(j > 0)
    def _():
        mx_ref[pl.ds(h, 1), :] = jnp.maximum(mx_ref[pl.ds(h, 1), :], pm)
        ms_ref[pl.ds(h, 1), :] = ms_ref[pl.ds(h, 1), :] + ps

    @pl.when(j == nj - 1)
    def _():
        o_ref[pl.ds(h, 1), :] = (mx_ref[pl.ds(h, 1), :]
                                 - ms_ref[pl.ds(h, 1), :] * (1.0 / _L))


def _mscore(qkv, qt, cnt_t):
    bj = 1024
    nj = _L // bj
    return pl.pallas_call(
        functools.partial(_mscore_kern, nj=nj),
        grid=(nj, _H),
        in_specs=[
            pl.BlockSpec((1, 1, bj, _DH), lambda j, h: (1, h, j, 0)),
            pl.BlockSpec((1, _DH, _L), lambda j, h: (h, 0, 0)),
            pl.BlockSpec((bj, _L), lambda j, h: (j, 0)),
        ],
        out_specs=pl.BlockSpec((_H, _L), lambda j, h: (0, 0)),
        out_shape=jax.ShapeDtypeStruct((_H, _L), jnp.float32),
        scratch_shapes=[pltpu.VMEM((_H, _L), jnp.float32),
                        pltpu.VMEM((_H, _L), jnp.float32)],
    )(qkv, qt, cnt_t)


def _topk_kern(m_ref, o_ref):
    m = m_ref[...]
    iota = jax.lax.broadcasted_iota(jnp.int32, (_H, _L), 1)
    for t in range(_U):
        cur = jnp.max(m, axis=1, keepdims=True)
        idx = jnp.min(jnp.where(m == cur, iota, _L), axis=1, keepdims=True)
        o_ref[:, t:t + 1] = idx
        m = jnp.where(iota == idx, _NEG, m)


def _topk(m):
    return pl.pallas_call(
        _topk_kern,
        grid=(1,),
        in_specs=[pl.BlockSpec((_H, _L), lambda i: (0, 0))],
        out_specs=pl.BlockSpec((_H, _U), lambda i: (0, 0)),
        out_shape=jax.ShapeDtypeStruct((_H, _U), jnp.int32),
    )(m)


_HB = 4


def _selattn_kern(q_ref, kt_ref, v_ref, tc_ref, tr_ref, o_ref):
    iota_c = jax.lax.broadcasted_iota(jnp.int32, (_U, _L), 1)
    iota_r = jax.lax.broadcasted_iota(jnp.int32, (_L, _U), 0)
    scs = []
    for hh in range(_HB):
        sel = (iota_c == tc_ref[0, hh]).astype(jnp.float32)
        qr = jnp.dot(sel, q_ref[0, hh], preferred_element_type=jnp.float32)
        scs.append(jnp.dot(qr, kt_ref[hh], preferred_element_type=jnp.float32))
    sc = jnp.concatenate(scs, axis=0) * (1.0 / 8.0)
    sc = sc - jnp.max(sc, axis=1, keepdims=True)
    e = jnp.exp(sc)
    attn = e / jnp.sum(e, axis=1, keepdims=True)
    for hh in range(_HB):
        a = attn[hh * _U:(hh + 1) * _U]
        v = v_ref[0, hh]
        upd = jnp.dot(a, v, preferred_element_type=jnp.float32)
        meanv = jnp.mean(v, axis=0, keepdims=True)
        sel_t = (iota_r == tr_ref[0, hh]).astype(jnp.float32)
        o_ref[hh] = meanv + jnp.dot(sel_t, upd - meanv,
                                    preferred_element_type=jnp.float32)


def _selattn(qkv, kt, tid_c, tid_r):
    return pl.pallas_call(
        _selattn_kern,
        grid=(_H // _HB,),
        in_specs=[
            pl.BlockSpec((1, _HB, _L, _DH), lambda h: (0, h, 0, 0)),
            pl.BlockSpec((_HB, _DH, _L), lambda h: (h, 0, 0)),
            pl.BlockSpec((1, _HB, _L, _DH), lambda h: (2, h, 0, 0)),
            pl.BlockSpec((1, _HB, _U, 1), lambda h: (0, h, 0, 0)),
            pl.BlockSpec((1, _HB, 1, _U), lambda h: (0, h, 0, 0)),
        ],
        out_specs=pl.BlockSpec((_HB, _L, _DH), lambda h: (h, 0, 0)),
        out_shape=jax.ShapeDtypeStruct((_H, _L, _DH), jnp.float32),
    )(qkv, kt, qkv, tid_c, tid_r)


def _addln_kern(x_ref, y_ref, g_ref, b_ref, o_ref):
    t = x_ref[...] + y_ref[...]
    mu = jnp.mean(t, axis=1, keepdims=True)
    d = t - mu
    var = jnp.mean(d * d, axis=1, keepdims=True)
    o_ref[...] = d * jax.lax.rsqrt(var + 1e-5) * g_ref[...] + b_ref[...]


def _addln(x, y, g, b):
    m = x.shape[0]
    bm = min(256, m)
    return pl.pallas_call(
        _addln_kern,
        grid=(m // bm,),
        in_specs=[
            pl.BlockSpec((bm, _DM), lambda i: (i, 0)),
            pl.BlockSpec((bm, _DM), lambda i: (i, 0)),
            pl.BlockSpec((1, _DM), lambda i: (0, 0)),
            pl.BlockSpec((1, _DM), lambda i: (0, 0)),
        ],
        out_specs=pl.BlockSpec((bm, _DM), lambda i: (i, 0)),
        out_shape=jax.ShapeDtypeStruct((m, _DM), jnp.float32),
    )(x, y, g.reshape(1, _DM), b.reshape(1, _DM))


def _final_kern(x_ref, g_ref, b_ref, wt_ref, o_ref):
    t = x_ref[...]
    mu = jnp.mean(t, axis=1, keepdims=True)
    d = t - mu
    var = jnp.mean(d * d, axis=1, keepdims=True)
    n = d * jax.lax.rsqrt(var + 1e-5) * g_ref[...] + b_ref[...]
    o_ref[...] = jnp.dot(n, wt_ref[...], preferred_element_type=jnp.float32)


def _final(x8, g, b, emb_t):
    return pl.pallas_call(
        _final_kern,
        grid=(1,),
        in_specs=[
            pl.BlockSpec((8, _DM), lambda i: (0, 0)),
            pl.BlockSpec((1, _DM), lambda i: (0, 0)),
            pl.BlockSpec((1, _DM), lambda i: (0, 0)),
            pl.BlockSpec((_DM, _NSTATE), lambda i: (0, 0)),
        ],
        out_specs=pl.BlockSpec((8, _NSTATE), lambda i: (0, 0)),
        out_shape=jax.ShapeDtypeStruct((8, _NSTATE), jnp.float32),
    )(x8, g.reshape(1, _DM), b.reshape(1, _DM), emb_t)


def _attention_ctx(x, p, cnt_t):
    wqkv = jnp.concatenate([p['Wq'], p['Wk'], p['Wv']], axis=1)
    bqkv = jnp.concatenate([p['bq'], p['bk'], p['bv']])
    qkv = _qkv(x, wqkv, bqkv)
    qt = jnp.swapaxes(qkv[0], 1, 2)
    kt = jnp.swapaxes(qkv[1], 1, 2)
    m = _mscore(qkv, qt, cnt_t)
    tid = _topk(m)
    ctx = _selattn(qkv, kt, tid.reshape(1, _H, _U, 1),
                   tid.reshape(1, _H, 1, _U))
    return ctx


def _layer_full(x, p, cnt_t):
    ctx = _attention_ctx(x, p, cnt_t)
    x1 = _attnout_ln(ctx, p['Wo'], p['bo'], x, p['g1'], p['be1'])
    h1 = _ffn1(x1, p['W1'], p['b1'])
    return _ffn2_ln(h1, p['W2'], p['b2'], x1, p['g2'], p['be2'])


def _layer_last8(x, p, cnt_t):
    ctx = _attention_ctx(x, p, cnt_t)
    ctx8 = ctx[:, _L - 8:, :].transpose(1, 0, 2).reshape(8, _DM)
    x8 = x[_L - 8:]
    attn8 = _matmul(ctx8, p['Wo'], p['bo'], bm=8)
    x1 = _addln(x8, attn8, p['g1'], p['be1'])
    h1 = _matmul(x1, p['W1'], p['b1'], act="gelu", bm=8)
    h2 = _matmul(h1, p['W2'], p['b2'], bm=8)
    return _addln(x1, h2, p['g2'], p['be2'])


def kernel(x_state_enc, x_enc, x_mark_enc, params):
    p = params
    st = x_state_enc.reshape(_L, 1).astype(jnp.int32)
    xe = x_enc.reshape(_L, 1).astype(jnp.float32)
    xm = x_mark_enc.reshape(_L, 4).astype(jnp.float32)

    bias0 = (p['b_val'] + p['b_mark']).reshape(1, _DM)
    pe = jnp.asarray(_PE)

    gathered = _sc_gather_rows(p['state_emb'], st.reshape(_L))
    x = _embed(gathered, xe, xm, pe, p['W_val'], p['W_mark'], bias0)
    x = _layer_full(x, p['layers'][0], jnp.asarray(_CNT_T[0]))
    x8 = _layer_last8(x, p['layers'][1], jnp.asarray(_CNT_T[1]))
    y = _final(x8, p['norm_g'], p['norm_b'], p['state_emb'].T)
    return y[7:8, :_NSTATE - 1].reshape(1, 1, _NSTATE - 1)


# speedup vs baseline: 1.2531x; 1.0716x over previous
import functools

import numpy as np
import jax
import jax.numpy as jnp
from jax.experimental import pallas as pl
from jax.experimental.pallas import tpu as pltpu
from jax.experimental.pallas import tpu_sc as plsc

_L = 2048
_DM = 1024
_H = 16
_DH = 64
_DFF = 4096
_NSTATE = 65
_FACTOR = 5
_U = min(int(_FACTOR * np.ceil(np.log(_L))), _L)
_NEG = -1e30


def _pe_table():
    pe = np.zeros((_L, _DM), np.float32)
    pos = np.arange(_L, dtype=np.float32)[:, None]
    div = np.exp(np.arange(0, _DM, 2, dtype=np.float32) * (-np.log(10000.0) / _DM))
    pe[:, 0::2] = np.sin(pos * div)
    pe[:, 1::2] = np.cos(pos * div)
    return pe


_PE = _pe_table()


def _sample_count_matrix_t(layer_idx):
    rng = jax.random.fold_in(jax.random.key(42), layer_idx)
    idx = np.asarray(jax.random.randint(rng, (_L, _U), 0, _L))
    cnt_t = np.zeros((_L, _L), np.float32)
    np.add.at(cnt_t, (idx, np.arange(_L)[:, None]), 1.0)
    return cnt_t


_CNT_T = [_sample_count_matrix_t(li) for li in range(2)]


def _mm_kern(x_ref, w_ref, b_ref, o_ref, acc_ref, *, nk, act):
    @pl.when(pl.program_id(2) == 0)
    def _():
        acc_ref[...] = jnp.zeros_like(acc_ref)

    acc_ref[...] += jnp.dot(x_ref[...], w_ref[...],
                            preferred_element_type=jnp.float32)

    @pl.when(pl.program_id(2) == nk - 1)
    def _():
        r = acc_ref[...] + b_ref[...]
        if act == "gelu":
            r = jax.nn.gelu(r)
        o_ref[...] = r


def _matmul(x, w, b, act=None, bm=256, bn=512, bk=512):
    m, k = x.shape
    n = w.shape[1]
    bm, bn, bk = min(bm, m), min(bn, n), min(bk, k)
    grid = (m // bm, n // bn, k // bk)
    return pl.pallas_call(
        functools.partial(_mm_kern, nk=grid[2], act=act),
        grid=grid,
        in_specs=[
            pl.BlockSpec((bm, bk), lambda i, j, kk: (i, kk)),
            pl.BlockSpec((bk, bn), lambda i, j, kk: (kk, j)),
            pl.BlockSpec((1, bn), lambda i, j, kk: (0, j)),
        ],
        out_specs=pl.BlockSpec((bm, bn), lambda i, j, kk: (i, j)),
        out_shape=jax.ShapeDtypeStruct((m, n), jnp.float32),
        scratch_shapes=[pltpu.VMEM((bm, bn), jnp.float32)],
    )(x, w, b.reshape(1, n))


def _qkv_kern(x_ref, w_ref, b_ref, o_ref):
    acc = jnp.dot(x_ref[...], w_ref[...],
                  preferred_element_type=jnp.float32) + b_ref[...]
    o_ref[0, 0] = acc[:, :_DH]
    o_ref[0, 1] = acc[:, _DH:]


def _qkv(x, wqkv, bqkv):
    bm = 2048
    grid = (_L // bm, 3, _H // 2)
    return pl.pallas_call(
        _qkv_kern,
        grid=grid,
        in_specs=[
            pl.BlockSpec((bm, _DM), lambda i, t, j: (i, 0)),
            pl.BlockSpec((_DM, 2 * _DH), lambda i, t, j: (0, t * (_H // 2) + j)),
            pl.BlockSpec((1, 2 * _DH), lambda i, t, j: (0, t * (_H // 2) + j)),
        ],
        out_specs=pl.BlockSpec((1, 2, bm, _DH), lambda i, t, j: (t, j, i, 0)),
        out_shape=jax.ShapeDtypeStruct((3, _H, _L, _DH), jnp.float32),
    )(x, wqkv, bqkv.reshape(1, 3 * _DM))


def _attnout_ln_kern(ctx_ref, w_ref, b_ref, res_ref, g_ref, be_ref, o_ref,
                     acc_ref, *, nk):
    kk = pl.program_id(1)

    @pl.when(kk == 0)
    def _():
        acc_ref[...] = jnp.zeros_like(acc_ref)

    c = ctx_ref[...]
    xcat = jnp.concatenate([c[0], c[1]], axis=1)
    acc_ref[...] += jnp.dot(xcat, w_ref[...], preferred_element_type=jnp.float32)

    @pl.when(kk == nk - 1)
    def _():
        t = acc_ref[...] + b_ref[...] + res_ref[...]
        mu = jnp.mean(t, axis=1, keepdims=True)
        d = t - mu
        var = jnp.mean(d * d, axis=1, keepdims=True)
        o_ref[...] = d * jax.lax.rsqrt(var + 1e-5) * g_ref[...] + be_ref[...]


def _attnout_ln(ctx, wo, bo, res, g, be):
    bm = 256
    nk = _H // 2
    return pl.pallas_call(
        functools.partial(_attnout_ln_kern, nk=nk),
        grid=(_L // bm, nk),
        in_specs=[
            pl.BlockSpec((2, bm, _DH), lambda i, kk: (kk, i, 0)),
            pl.BlockSpec((2 * _DH, _DM), lambda i, kk: (kk, 0)),
            pl.BlockSpec((1, _DM), lambda i, kk: (0, 0)),
            pl.BlockSpec((bm, _DM), lambda i, kk: (i, 0)),
            pl.BlockSpec((1, _DM), lambda i, kk: (0, 0)),
            pl.BlockSpec((1, _DM), lambda i, kk: (0, 0)),
        ],
        out_specs=pl.BlockSpec((bm, _DM), lambda i, kk: (i, 0)),
        out_shape=jax.ShapeDtypeStruct((_L, _DM), jnp.float32),
        scratch_shapes=[pltpu.VMEM((bm, _DM), jnp.float32)],
    )(ctx, wo, bo.reshape(1, _DM), res, g.reshape(1, _DM), be.reshape(1, _DM))


def _ffn1_kern(x_ref, w_ref, b_ref, o_ref):
    o_ref[...] = jax.nn.gelu(
        jnp.dot(x_ref[...], w_ref[...], preferred_element_type=jnp.float32)
        + b_ref[...])


def _ffn1(x, w1, b1):
    bm, bn = 2048, 512
    return pl.pallas_call(
        _ffn1_kern,
        grid=(_L // bm, _DFF // bn),
        in_specs=[
            pl.BlockSpec((bm, _DM), lambda i, j: (i, 0)),
            pl.BlockSpec((_DM, bn), lambda i, j: (0, j)),
            pl.BlockSpec((1, bn), lambda i, j: (0, j)),
        ],
        out_specs=pl.BlockSpec((bm, bn), lambda i, j: (i, j)),
        out_shape=jax.ShapeDtypeStruct((_L, _DFF), jnp.float32),
    )(x, w1, b1.reshape(1, _DFF))


def _ffn2_ln_kern(x_ref, w_ref, b_ref, res_ref, g_ref, be_ref, o_ref,
                  acc_ref, *, nk):
    kk = pl.program_id(1)

    @pl.when(kk == 0)
    def _():
        acc_ref[...] = jnp.zeros_like(acc_ref)

    acc_ref[...] += jnp.dot(x_ref[...], w_ref[...],
                            preferred_element_type=jnp.float32)

    @pl.when(kk == nk - 1)
    def _():
        t = acc_ref[...] + b_ref[...] + res_ref[...]
        mu = jnp.mean(t, axis=1, keepdims=True)
        d = t - mu
        var = jnp.mean(d * d, axis=1, keepdims=True)
        o_ref[...] = d * jax.lax.rsqrt(var + 1e-5) * g_ref[...] + be_ref[...]


def _ffn2_ln(h1, w2, b2, res, g, be):
    bm, bk = 512, 512
    nk = _DFF // bk
    return pl.pallas_call(
        functools.partial(_ffn2_ln_kern, nk=nk),
        grid=(_L // bm, nk),
        in_specs=[
            pl.BlockSpec((bm, bk), lambda i, kk: (i, kk)),
            pl.BlockSpec((bk, _DM), lambda i, kk: (kk, 0)),
            pl.BlockSpec((1, _DM), lambda i, kk: (0, 0)),
            pl.BlockSpec((bm, _DM), lambda i, kk: (i, 0)),
            pl.BlockSpec((1, _DM), lambda i, kk: (0, 0)),
            pl.BlockSpec((1, _DM), lambda i, kk: (0, 0)),
        ],
        out_specs=pl.BlockSpec((bm, _DM), lambda i, kk: (i, 0)),
        out_shape=jax.ShapeDtypeStruct((_L, _DM), jnp.float32),
        scratch_shapes=[pltpu.VMEM((bm, _DM), jnp.float32)],
    )(h1, w2, b2.reshape(1, _DM), res, g.reshape(1, _DM), be.reshape(1, _DM))


_NC, _NS = 2, 16
_NW = _NC * _NS
_BPW = _L // _NW


def _sc_gather_rows(table, idx):
    mesh = plsc.VectorSubcoreMesh(core_axis_name="c", subcore_axis_name="s")

    @functools.partial(
        pl.kernel, mesh=mesh,
        out_type=jax.ShapeDtypeStruct((_L, _DM), jnp.float32),
        scratch_types=[
            pltpu.VMEM((_BPW,), jnp.int32),
            pltpu.VMEM((_BPW, _DM), jnp.float32),
            pltpu.SemaphoreType.DMA,
        ],
    )
    def k(table_hbm, idx_hbm, out_hbm, idx_v, rows_v, sem):
        wid = jax.lax.axis_index("s") * _NC + jax.lax.axis_index("c")
        base = wid * _BPW
        pltpu.sync_copy(idx_hbm.at[pl.ds(base, _BPW)], idx_v)
        pltpu.async_copy(table_hbm.at[idx_v], rows_v, sem).wait()
        pltpu.sync_copy(rows_v, out_hbm.at[pl.ds(base, _BPW)])

    return k(table, idx)


def _embed_kern(g_ref, xe_ref, xm_ref, pe_ref, wv_ref, wm_ref, b_ref, o_ref):
    r = g_ref[...]
    r += xe_ref[...] * wv_ref[...]
    r += jnp.dot(xm_ref[...], wm_ref[...], preferred_element_type=jnp.float32)
    o_ref[...] = r + b_ref[...] + pe_ref[...]


def _embed(gathered, xe, xm, pe, wv, wm, bias):
    bm = 256
    grid = (_L // bm,)
    return pl.pallas_call(
        _embed_kern,
        grid=grid,
        in_specs=[
            pl.BlockSpec((bm, _DM), lambda i: (i, 0)),
            pl.BlockSpec((bm, 1), lambda i: (i, 0)),
            pl.BlockSpec((bm, 4), lambda i: (i, 0)),
            pl.BlockSpec((bm, _DM), lambda i: (i, 0)),
            pl.BlockSpec((1, _DM), lambda i: (0, 0)),
            pl.BlockSpec((4, _DM), lambda i: (0, 0)),
            pl.BlockSpec((1, _DM), lambda i: (0, 0)),
        ],
        out_specs=pl.BlockSpec((bm, _DM), lambda i: (i, 0)),
        out_shape=jax.ShapeDtypeStruct((_L, _DM), jnp.float32),
    )(gathered, xe, xm, pe, wv, wm, bias)


def _mscore_kern(k_ref, qt_ref, cnt_ref, o_ref, mx_ref, ms_ref, *, nj):
    j = pl.program_id(0)
    h = pl.program_id(1)
    kk = k_ref[0, 0]
    qt = qt_ref[0]
    s = jnp.dot(kk, qt, preferred_element_type=jnp.float32)
    c = cnt_ref[...]
    pm = jnp.max(jnp.where(c > 0.0, s, _NEG), axis=0, keepdims=True)
    ps = jnp.sum(s * c, axis=0, keepdims=True)

    @pl.when(j == 0)
    def _():
        mx_ref[pl.ds(h, 1), :] = pm
        ms_ref[pl.ds(h, 1), :] = ps

    @pl.when(j > 0)
    def _():
        mx_ref[pl.ds(h, 1), :] = jnp.maximum(mx_ref[pl.ds(h, 1), :], pm)
        ms_ref[pl.ds(h, 1), :] = ms_ref[pl.ds(h, 1), :] + ps

    @pl.when(j == nj - 1)
    def _():
        o_ref[pl.ds(h, 1), :] = (mx_ref[pl.ds(h, 1), :]
                                 - ms_ref[pl.ds(h, 1), :] * (1.0 / _L))


def _mscore(qkv, qt, cnt_t):
    bj = 1024
    nj = _L // bj
    return pl.pallas_call(
        functools.partial(_mscore_kern, nj=nj),
        grid=(nj, _H),
        in_specs=[
            pl.BlockSpec((1, 1, bj, _DH), lambda j, h: (1, h, j, 0)),
            pl.BlockSpec((1, _DH, _L), lambda j, h: (h, 0, 0)),
            pl.BlockSpec((bj, _L), lambda j, h: (j, 0)),
        ],
        out_specs=pl.BlockSpec((_H, _L), lambda j, h: (0, 0)),
        out_shape=jax.ShapeDtypeStruct((_H, _L), jnp.float32),
        scratch_shapes=[pltpu.VMEM((_H, _L), jnp.float32),
                        pltpu.VMEM((_H, _L), jnp.float32)],
    )(qkv, qt, cnt_t)


def _topk_kern(m_ref, o_ref):
    m = m_ref[...]
    iota = jax.lax.broadcasted_iota(jnp.int32, (_H, _L), 1)
    for t in range(_U):
        cur = jnp.max(m, axis=1, keepdims=True)
        idx = jnp.min(jnp.where(m == cur, iota, _L), axis=1, keepdims=True)
        o_ref[:, t:t + 1] = idx
        m = jnp.where(iota == idx, _NEG, m)


def _topk(m):
    return pl.pallas_call(
        _topk_kern,
        grid=(1,),
        in_specs=[pl.BlockSpec((_H, _L), lambda i: (0, 0))],
        out_specs=pl.BlockSpec((_H, _U), lambda i: (0, 0)),
        out_shape=jax.ShapeDtypeStruct((_H, _U), jnp.int32),
    )(m)


_HB = 4


def _selattn_kern(q_ref, kt_ref, v_ref, tc_ref, tr_ref, o_ref):
    iota_c = jax.lax.broadcasted_iota(jnp.int32, (_U, _L), 1)
    iota_r = jax.lax.broadcasted_iota(jnp.int32, (_L, _U), 0)
    scs = []
    for hh in range(_HB):
        sel = (iota_c == tc_ref[0, hh]).astype(jnp.float32)
        qr = jnp.dot(sel, q_ref[0, hh], preferred_element_type=jnp.float32)
        scs.append(jnp.dot(qr, kt_ref[hh], preferred_element_type=jnp.float32))
    sc = jnp.concatenate(scs, axis=0) * (1.0 / 8.0)
    sc = sc - jnp.max(sc, axis=1, keepdims=True)
    e = jnp.exp(sc)
    attn = e / jnp.sum(e, axis=1, keepdims=True)
    for hh in range(_HB):
        a = attn[hh * _U:(hh + 1) * _U]
        v = v_ref[0, hh]
        upd = jnp.dot(a, v, preferred_element_type=jnp.float32)
        meanv = jnp.mean(v, axis=0, keepdims=True)
        sel_t = (iota_r == tr_ref[0, hh]).astype(jnp.float32)
        o_ref[hh] = meanv + jnp.dot(sel_t, upd - meanv,
                                    preferred_element_type=jnp.float32)


def _selattn(qkv, kt, tid_c, tid_r):
    return pl.pallas_call(
        _selattn_kern,
        grid=(_H // _HB,),
        in_specs=[
            pl.BlockSpec((1, _HB, _L, _DH), lambda h: (0, h, 0, 0)),
            pl.BlockSpec((_HB, _DH, _L), lambda h: (h, 0, 0)),
            pl.BlockSpec((1, _HB, _L, _DH), lambda h: (2, h, 0, 0)),
            pl.BlockSpec((1, _HB, _U, 1), lambda h: (0, h, 0, 0)),
            pl.BlockSpec((1, _HB, 1, _U), lambda h: (0, h, 0, 0)),
        ],
        out_specs=pl.BlockSpec((_HB, _L, _DH), lambda h: (h, 0, 0)),
        out_shape=jax.ShapeDtypeStruct((_H, _L, _DH), jnp.float32),
    )(qkv, kt, qkv, tid_c, tid_r)


def _addln_kern(x_ref, y_ref, g_ref, b_ref, o_ref):
    t = x_ref[...] + y_ref[...]
    mu = jnp.mean(t, axis=1, keepdims=True)
    d = t - mu
    var = jnp.mean(d * d, axis=1, keepdims=True)
    o_ref[...] = d * jax.lax.rsqrt(var + 1e-5) * g_ref[...] + b_ref[...]


def _addln(x, y, g, b):
    m = x.shape[0]
    bm = min(256, m)
    return pl.pallas_call(
        _addln_kern,
        grid=(m // bm,),
        in_specs=[
            pl.BlockSpec((bm, _DM), lambda i: (i, 0)),
            pl.BlockSpec((bm, _DM), lambda i: (i, 0)),
            pl.BlockSpec((1, _DM), lambda i: (0, 0)),
            pl.BlockSpec((1, _DM), lambda i: (0, 0)),
        ],
        out_specs=pl.BlockSpec((bm, _DM), lambda i: (i, 0)),
        out_shape=jax.ShapeDtypeStruct((m, _DM), jnp.float32),
    )(x, y, g.reshape(1, _DM), b.reshape(1, _DM))


def _final_kern(x_ref, g_ref, b_ref, wt_ref, o_ref):
    t = x_ref[...]
    mu = jnp.mean(t, axis=1, keepdims=True)
    d = t - mu
    var = jnp.mean(d * d, axis=1, keepdims=True)
    n = d * jax.lax.rsqrt(var + 1e-5) * g_ref[...] + b_ref[...]
    o_ref[...] = jnp.dot(n, wt_ref[...], preferred_element_type=jnp.float32)


def _final(x8, g, b, emb_t):
    return pl.pallas_call(
        _final_kern,
        grid=(1,),
        in_specs=[
            pl.BlockSpec((8, _DM), lambda i: (0, 0)),
            pl.BlockSpec((1, _DM), lambda i: (0, 0)),
            pl.BlockSpec((1, _DM), lambda i: (0, 0)),
            pl.BlockSpec((_DM, _NSTATE), lambda i: (0, 0)),
        ],
        out_specs=pl.BlockSpec((8, _NSTATE), lambda i: (0, 0)),
        out_shape=jax.ShapeDtypeStruct((8, _NSTATE), jnp.float32),
    )(x8, g.reshape(1, _DM), b.reshape(1, _DM), emb_t)


def _attention_ctx(x, p, cnt_t):
    wqkv = jnp.concatenate([p['Wq'], p['Wk'], p['Wv']], axis=1)
    bqkv = jnp.concatenate([p['bq'], p['bk'], p['bv']])
    qkv = _qkv(x, wqkv, bqkv)
    qt = jnp.swapaxes(qkv[0], 1, 2)
    kt = jnp.swapaxes(qkv[1], 1, 2)
    m = _mscore(qkv, qt, cnt_t)
    tid = _topk(m)
    ctx = _selattn(qkv, kt, tid.reshape(1, _H, _U, 1),
                   tid.reshape(1, _H, 1, _U))
    return ctx


def _layer_full(x, p, cnt_t):
    ctx = _attention_ctx(x, p, cnt_t)
    x1 = _attnout_ln(ctx, p['Wo'], p['bo'], x, p['g1'], p['be1'])
    h1 = _ffn1(x1, p['W1'], p['b1'])
    return _ffn2_ln(h1, p['W2'], p['b2'], x1, p['g2'], p['be2'])


def _layer_last8(x, p, cnt_t):
    ctx = _attention_ctx(x, p, cnt_t)
    ctx8 = ctx[:, _L - 8:, :].transpose(1, 0, 2).reshape(8, _DM)
    x8 = x[_L - 8:]
    attn8 = _matmul(ctx8, p['Wo'], p['bo'], bm=8)
    x1 = _addln(x8, attn8, p['g1'], p['be1'])
    h1 = _matmul(x1, p['W1'], p['b1'], act="gelu", bm=8)
    h2 = _matmul(h1, p['W2'], p['b2'], bm=8)
    return _addln(x1, h2, p['g2'], p['be2'])


def kernel(x_state_enc, x_enc, x_mark_enc, params):
    p = params
    st = x_state_enc.reshape(_L, 1).astype(jnp.int32)
    xe = x_enc.reshape(_L, 1).astype(jnp.float32)
    xm = x_mark_enc.reshape(_L, 4).astype(jnp.float32)

    bias0 = (p['b_val'] + p['b_mark']).reshape(1, _DM)
    pe = jnp.asarray(_PE)

    gathered = _sc_gather_rows(p['state_emb'], st.reshape(_L))
    x = _embed(gathered, xe, xm, pe, p['W_val'], p['W_mark'], bias0)
    x = _layer_full(x, p['layers'][0], jnp.asarray(_CNT_T[0]))
    x8 = _layer_last8(x, p['layers'][1], jnp.asarray(_CNT_T[1]))
    y = _final(x8, p['norm_g'], p['norm_b'], p['state_emb'].T)
    return y[7:8, :_NSTATE - 1].reshape(1, 1, _NSTATE - 1)


# speedup vs baseline: 1.3295x; 1.0610x over previous
import functools

import numpy as np
import jax
import jax.numpy as jnp
from jax.experimental import pallas as pl
from jax.experimental.pallas import tpu as pltpu
from jax.experimental.pallas import tpu_sc as plsc

_L = 2048
_DM = 1024
_H = 16
_DH = 64
_DFF = 4096
_NSTATE = 65
_FACTOR = 5
_U = min(int(_FACTOR * np.ceil(np.log(_L))), _L)
_NEG = -1e30


def _pe_table():
    pe = np.zeros((_L, _DM), np.float32)
    pos = np.arange(_L, dtype=np.float32)[:, None]
    div = np.exp(np.arange(0, _DM, 2, dtype=np.float32) * (-np.log(10000.0) / _DM))
    pe[:, 0::2] = np.sin(pos * div)
    pe[:, 1::2] = np.cos(pos * div)
    return pe


_PE = _pe_table()


def _sample_count_matrix_t(layer_idx):
    rng = jax.random.fold_in(jax.random.key(42), layer_idx)
    idx = np.asarray(jax.random.randint(rng, (_L, _U), 0, _L))
    cnt_t = np.zeros((_L, _L), np.float32)
    np.add.at(cnt_t, (idx, np.arange(_L)[:, None]), 1.0)
    return cnt_t


_CNT_T = [_sample_count_matrix_t(li) for li in range(2)]


def _mm_kern(x_ref, w_ref, b_ref, o_ref, acc_ref, *, nk, act):
    @pl.when(pl.program_id(2) == 0)
    def _():
        acc_ref[...] = jnp.zeros_like(acc_ref)

    acc_ref[...] += jnp.dot(x_ref[...], w_ref[...],
                            preferred_element_type=jnp.float32)

    @pl.when(pl.program_id(2) == nk - 1)
    def _():
        r = acc_ref[...] + b_ref[...]
        if act == "gelu":
            r = jax.nn.gelu(r)
        o_ref[...] = r


def _matmul(x, w, b, act=None, bm=256, bn=512, bk=512):
    m, k = x.shape
    n = w.shape[1]
    bm, bn, bk = min(bm, m), min(bn, n), min(bk, k)
    grid = (m // bm, n // bn, k // bk)
    return pl.pallas_call(
        functools.partial(_mm_kern, nk=grid[2], act=act),
        grid=grid,
        in_specs=[
            pl.BlockSpec((bm, bk), lambda i, j, kk: (i, kk)),
            pl.BlockSpec((bk, bn), lambda i, j, kk: (kk, j)),
            pl.BlockSpec((1, bn), lambda i, j, kk: (0, j)),
        ],
        out_specs=pl.BlockSpec((bm, bn), lambda i, j, kk: (i, j)),
        out_shape=jax.ShapeDtypeStruct((m, n), jnp.float32),
        scratch_shapes=[pltpu.VMEM((bm, bn), jnp.float32)],
    )(x, w, b.reshape(1, n))


def _qkv_kern(x_ref, w_ref, b_ref, o_ref):
    acc = jnp.dot(x_ref[...], w_ref[...],
                  preferred_element_type=jnp.float32) + b_ref[...]
    o_ref[0, 0] = acc[:, :_DH]
    o_ref[0, 1] = acc[:, _DH:]


def _qkv(x, wqkv, bqkv):
    bm = 2048
    grid = (_L // bm, 3, _H // 2)
    return pl.pallas_call(
        _qkv_kern,
        grid=grid,
        in_specs=[
            pl.BlockSpec((bm, _DM), lambda i, t, j: (i, 0)),
            pl.BlockSpec((_DM, 2 * _DH), lambda i, t, j: (0, t * (_H // 2) + j)),
            pl.BlockSpec((1, 2 * _DH), lambda i, t, j: (0, t * (_H // 2) + j)),
        ],
        out_specs=pl.BlockSpec((1, 2, bm, _DH), lambda i, t, j: (t, j, i, 0)),
        out_shape=jax.ShapeDtypeStruct((3, _H, _L, _DH), jnp.float32),
    )(x, wqkv, bqkv.reshape(1, 3 * _DM))


def _attnout_ln_kern(ctx_ref, w_ref, b_ref, res_ref, g_ref, be_ref, o_ref,
                     acc_ref, *, nk):
    kk = pl.program_id(1)

    @pl.when(kk == 0)
    def _():
        acc_ref[...] = jnp.zeros_like(acc_ref)

    c = ctx_ref[...]
    xcat = jnp.concatenate([c[0], c[1]], axis=1)
    acc_ref[...] += jnp.dot(xcat, w_ref[...], preferred_element_type=jnp.float32)

    @pl.when(kk == nk - 1)
    def _():
        t = acc_ref[...] + b_ref[...] + res_ref[...]
        mu = jnp.mean(t, axis=1, keepdims=True)
        d = t - mu
        var = jnp.mean(d * d, axis=1, keepdims=True)
        o_ref[...] = d * jax.lax.rsqrt(var + 1e-5) * g_ref[...] + be_ref[...]


def _attnout_ln(ctx, wo, bo, res, g, be):
    bm = 512
    nk = _H // 2
    return pl.pallas_call(
        functools.partial(_attnout_ln_kern, nk=nk),
        grid=(_L // bm, nk),
        in_specs=[
            pl.BlockSpec((2, bm, _DH), lambda i, kk: (kk, i, 0)),
            pl.BlockSpec((2 * _DH, _DM), lambda i, kk: (kk, 0)),
            pl.BlockSpec((1, _DM), lambda i, kk: (0, 0)),
            pl.BlockSpec((bm, _DM), lambda i, kk: (i, 0)),
            pl.BlockSpec((1, _DM), lambda i, kk: (0, 0)),
            pl.BlockSpec((1, _DM), lambda i, kk: (0, 0)),
        ],
        out_specs=pl.BlockSpec((bm, _DM), lambda i, kk: (i, 0)),
        out_shape=jax.ShapeDtypeStruct((_L, _DM), jnp.float32),
        scratch_shapes=[pltpu.VMEM((bm, _DM), jnp.float32)],
    )(ctx, wo, bo.reshape(1, _DM), res, g.reshape(1, _DM), be.reshape(1, _DM))


def _ffn1_kern(x_ref, w_ref, b_ref, o_ref):
    o_ref[...] = jax.nn.gelu(
        jnp.dot(x_ref[...], w_ref[...], preferred_element_type=jnp.float32)
        + b_ref[...])


def _ffn1(x, w1, b1):
    bm, bn = 2048, 512
    return pl.pallas_call(
        _ffn1_kern,
        grid=(_L // bm, _DFF // bn),
        in_specs=[
            pl.BlockSpec((bm, _DM), lambda i, j: (i, 0)),
            pl.BlockSpec((_DM, bn), lambda i, j: (0, j)),
            pl.BlockSpec((1, bn), lambda i, j: (0, j)),
        ],
        out_specs=pl.BlockSpec((bm, bn), lambda i, j: (i, j)),
        out_shape=jax.ShapeDtypeStruct((_L, _DFF), jnp.float32),
    )(x, w1, b1.reshape(1, _DFF))


def _ffn2_ln_kern(x_ref, w_ref, b_ref, res_ref, g_ref, be_ref, o_ref,
                  acc_ref, *, nk):
    kk = pl.program_id(1)

    @pl.when(kk == 0)
    def _():
        acc_ref[...] = jnp.zeros_like(acc_ref)

    acc_ref[...] += jnp.dot(x_ref[...], w_ref[...],
                            preferred_element_type=jnp.float32)

    @pl.when(kk == nk - 1)
    def _():
        t = acc_ref[...] + b_ref[...] + res_ref[...]
        mu = jnp.mean(t, axis=1, keepdims=True)
        d = t - mu
        var = jnp.mean(d * d, axis=1, keepdims=True)
        o_ref[...] = d * jax.lax.rsqrt(var + 1e-5) * g_ref[...] + be_ref[...]


def _ffn2_ln(h1, w2, b2, res, g, be):
    bm, bk = 1024, 512
    nk = _DFF // bk
    return pl.pallas_call(
        functools.partial(_ffn2_ln_kern, nk=nk),
        grid=(_L // bm, nk),
        in_specs=[
            pl.BlockSpec((bm, bk), lambda i, kk: (i, kk)),
            pl.BlockSpec((bk, _DM), lambda i, kk: (kk, 0)),
            pl.BlockSpec((1, _DM), lambda i, kk: (0, 0)),
            pl.BlockSpec((bm, _DM), lambda i, kk: (i, 0)),
            pl.BlockSpec((1, _DM), lambda i, kk: (0, 0)),
            pl.BlockSpec((1, _DM), lambda i, kk: (0, 0)),
        ],
        out_specs=pl.BlockSpec((bm, _DM), lambda i, kk: (i, 0)),
        out_shape=jax.ShapeDtypeStruct((_L, _DM), jnp.float32),
        scratch_shapes=[pltpu.VMEM((bm, _DM), jnp.float32)],
    )(h1, w2, b2.reshape(1, _DM), res, g.reshape(1, _DM), be.reshape(1, _DM))


_NC, _NS = 2, 16
_NW = _NC * _NS
_BPW = _L // _NW


def _sc_gather_rows(table, idx):
    mesh = plsc.VectorSubcoreMesh(core_axis_name="c", subcore_axis_name="s")

    @functools.partial(
        pl.kernel, mesh=mesh,
        out_type=jax.ShapeDtypeStruct((_L, _DM), jnp.float32),
        scratch_types=[
            pltpu.VMEM((_BPW,), jnp.int32),
            pltpu.VMEM((_BPW, _DM), jnp.float32),
            pltpu.SemaphoreType.DMA,
        ],
    )
    def k(table_hbm, idx_hbm, out_hbm, idx_v, rows_v, sem):
        wid = jax.lax.axis_index("s") * _NC + jax.lax.axis_index("c")
        base = wid * _BPW
        pltpu.sync_copy(idx_hbm.at[pl.ds(base, _BPW)], idx_v)
        pltpu.async_copy(table_hbm.at[idx_v], rows_v, sem).wait()
        pltpu.sync_copy(rows_v, out_hbm.at[pl.ds(base, _BPW)])

    return k(table, idx)


def _embed_kern(g_ref, xe_ref, xm_ref, pe_ref, wv_ref, wm_ref, b_ref, o_ref):
    r = g_ref[...]
    r += xe_ref[...] * wv_ref[...]
    r += jnp.dot(xm_ref[...], wm_ref[...], preferred_element_type=jnp.float32)
    o_ref[...] = r + b_ref[...] + pe_ref[...]


def _embed(gathered, xe, xm, pe, wv, wm, bias):
    bm = 256
    grid = (_L // bm,)
    return pl.pallas_call(
        _embed_kern,
        grid=grid,
        in_specs=[
            pl.BlockSpec((bm, _DM), lambda i: (i, 0)),
            pl.BlockSpec((bm, 1), lambda i: (i, 0)),
            pl.BlockSpec((bm, 4), lambda i: (i, 0)),
            pl.BlockSpec((bm, _DM), lambda i: (i, 0)),
            pl.BlockSpec((1, _DM), lambda i: (0, 0)),
            pl.BlockSpec((4, _DM), lambda i: (0, 0)),
            pl.BlockSpec((1, _DM), lambda i: (0, 0)),
        ],
        out_specs=pl.BlockSpec((bm, _DM), lambda i: (i, 0)),
        out_shape=jax.ShapeDtypeStruct((_L, _DM), jnp.float32),
    )(gathered, xe, xm, pe, wv, wm, bias)


def _mscore_kern(k_ref, qt_ref, cnt_ref, o_ref, mx_ref, ms_ref, *, nj):
    j = pl.program_id(0)
    h = pl.program_id(1)
    kk = k_ref[0, 0]
    qt = qt_ref[0]
    s = jnp.dot(kk, qt, preferred_element_type=jnp.float32)
    c = cnt_ref[...]
    pm = jnp.max(jnp.where(c > 0.0, s, _NEG), axis=0, keepdims=True)
    ps = jnp.sum(s * c, axis=0, keepdims=True)

    @pl.when(j == 0)
    def _():
        mx_ref[pl.ds(h, 1), :] = pm
        ms_ref[pl.ds(h, 1), :] = ps

    @pl.when(j > 0)
    def _():
        mx_ref[pl.ds(h, 1), :] = jnp.maximum(mx_ref[pl.ds(h, 1), :], pm)
        ms_ref[pl.ds(h, 1), :] = ms_ref[pl.ds(h, 1), :] + ps

    @pl.when(j == nj - 1)
    def _():
        o_ref[pl.ds(h, 1), :] = (mx_ref[pl.ds(h, 1), :]
                                 - ms_ref[pl.ds(h, 1), :] * (1.0 / _L))


def _mscore(qkv, qt, cnt_t):
    bj = 1024
    nj = _L // bj
    return pl.pallas_call(
        functools.partial(_mscore_kern, nj=nj),
        grid=(nj, _H),
        in_specs=[
            pl.BlockSpec((1, 1, bj, _DH), lambda j, h: (1, h, j, 0)),
            pl.BlockSpec((1, _DH, _L), lambda j, h: (h, 0, 0)),
            pl.BlockSpec((bj, _L), lambda j, h: (j, 0)),
        ],
        out_specs=pl.BlockSpec((_H, _L), lambda j, h: (0, 0)),
        out_shape=jax.ShapeDtypeStruct((_H, _L), jnp.float32),
        scratch_shapes=[pltpu.VMEM((_H, _L), jnp.float32),
                        pltpu.VMEM((_H, _L), jnp.float32)],
    )(qkv, qt, cnt_t)


def _topk_kern(m_ref, o_ref):
    m = m_ref[...]
    iota = jax.lax.broadcasted_iota(jnp.int32, (_H, _L), 1)
    for t in range(_U):
        cur = jnp.max(m, axis=1, keepdims=True)
        idx = jnp.min(jnp.where(m == cur, iota, _L), axis=1, keepdims=True)
        o_ref[:, t:t + 1] = idx
        m = jnp.where(iota == idx, _NEG, m)


def _topk(m):
    return pl.pallas_call(
        _topk_kern,
        grid=(1,),
        in_specs=[pl.BlockSpec((_H, _L), lambda i: (0, 0))],
        out_specs=pl.BlockSpec((_H, _U), lambda i: (0, 0)),
        out_shape=jax.ShapeDtypeStruct((_H, _U), jnp.int32),
    )(m)


_HB = 4


def _selattn_kern(q_ref, kt_ref, v_ref, tc_ref, tr_ref, o_ref):
    iota_c = jax.lax.broadcasted_iota(jnp.int32, (_U, _L), 1)
    iota_r = jax.lax.broadcasted_iota(jnp.int32, (_L, _U), 0)
    scs = []
    for hh in range(_HB):
        sel = (iota_c == tc_ref[0, hh]).astype(jnp.float32)
        qr = jnp.dot(sel, q_ref[0, hh], preferred_element_type=jnp.float32)
        scs.append(jnp.dot(qr, kt_ref[hh], preferred_element_type=jnp.float32))
    sc = jnp.concatenate(scs, axis=0) * (1.0 / 8.0)
    sc = sc - jnp.max(sc, axis=1, keepdims=True)
    e = jnp.exp(sc)
    attn = e / jnp.sum(e, axis=1, keepdims=True)
    for hh in range(_HB):
        a = attn[hh * _U:(hh + 1) * _U]
        v = v_ref[0, hh]
        upd = jnp.dot(a, v, preferred_element_type=jnp.float32)
        meanv = jnp.mean(v, axis=0, keepdims=True)
        sel_t = (iota_r == tr_ref[0, hh]).astype(jnp.float32)
        o_ref[hh] = meanv + jnp.dot(sel_t, upd - meanv,
                                    preferred_element_type=jnp.float32)


def _selattn(qkv, kt, tid_c, tid_r):
    return pl.pallas_call(
        _selattn_kern,
        grid=(_H // _HB,),
        in_specs=[
            pl.BlockSpec((1, _HB, _L, _DH), lambda h: (0, h, 0, 0)),
            pl.BlockSpec((_HB, _DH, _L), lambda h: (h, 0, 0)),
            pl.BlockSpec((1, _HB, _L, _DH), lambda h: (2, h, 0, 0)),
            pl.BlockSpec((1, _HB, _U, 1), lambda h: (0, h, 0, 0)),
            pl.BlockSpec((1, _HB, 1, _U), lambda h: (0, h, 0, 0)),
        ],
        out_specs=pl.BlockSpec((_HB, _L, _DH), lambda h: (h, 0, 0)),
        out_shape=jax.ShapeDtypeStruct((_H, _L, _DH), jnp.float32),
    )(qkv, kt, qkv, tid_c, tid_r)


def _addln_kern(x_ref, y_ref, g_ref, b_ref, o_ref):
    t = x_ref[...] + y_ref[...]
    mu = jnp.mean(t, axis=1, keepdims=True)
    d = t - mu
    var = jnp.mean(d * d, axis=1, keepdims=True)
    o_ref[...] = d * jax.lax.rsqrt(var + 1e-5) * g_ref[...] + b_ref[...]


def _addln(x, y, g, b):
    m = x.shape[0]
    bm = min(256, m)
    return pl.pallas_call(
        _addln_kern,
        grid=(m // bm,),
        in_specs=[
            pl.BlockSpec((bm, _DM), lambda i: (i, 0)),
            pl.BlockSpec((bm, _DM), lambda i: (i, 0)),
            pl.BlockSpec((1, _DM), lambda i: (0, 0)),
            pl.BlockSpec((1, _DM), lambda i: (0, 0)),
        ],
        out_specs=pl.BlockSpec((bm, _DM), lambda i: (i, 0)),
        out_shape=jax.ShapeDtypeStruct((m, _DM), jnp.float32),
    )(x, y, g.reshape(1, _DM), b.reshape(1, _DM))


def _final_kern(x_ref, g_ref, b_ref, wt_ref, o_ref):
    t = x_ref[...]
    mu = jnp.mean(t, axis=1, keepdims=True)
    d = t - mu
    var = jnp.mean(d * d, axis=1, keepdims=True)
    n = d * jax.lax.rsqrt(var + 1e-5) * g_ref[...] + b_ref[...]
    o_ref[...] = jnp.dot(n, wt_ref[...], preferred_element_type=jnp.float32)


def _final(x8, g, b, emb_t):
    return pl.pallas_call(
        _final_kern,
        grid=(1,),
        in_specs=[
            pl.BlockSpec((8, _DM), lambda i: (0, 0)),
            pl.BlockSpec((1, _DM), lambda i: (0, 0)),
            pl.BlockSpec((1, _DM), lambda i: (0, 0)),
            pl.BlockSpec((_DM, _NSTATE), lambda i: (0, 0)),
        ],
        out_specs=pl.BlockSpec((8, _NSTATE), lambda i: (0, 0)),
        out_shape=jax.ShapeDtypeStruct((8, _NSTATE), jnp.float32),
    )(x8, g.reshape(1, _DM), b.reshape(1, _DM), emb_t)


def _attention_ctx(x, p, cnt_t):
    wqkv = jnp.concatenate([p['Wq'], p['Wk'], p['Wv']], axis=1)
    bqkv = jnp.concatenate([p['bq'], p['bk'], p['bv']])
    qkv = _qkv(x, wqkv, bqkv)
    qt = jnp.swapaxes(qkv[0], 1, 2)
    kt = jnp.swapaxes(qkv[1], 1, 2)
    m = _mscore(qkv, qt, cnt_t)
    tid = _topk(m)
    ctx = _selattn(qkv, kt, tid.reshape(1, _H, _U, 1),
                   tid.reshape(1, _H, 1, _U))
    return ctx


def _layer_full(x, p, cnt_t):
    ctx = _attention_ctx(x, p, cnt_t)
    x1 = _attnout_ln(ctx, p['Wo'], p['bo'], x, p['g1'], p['be1'])
    h1 = _ffn1(x1, p['W1'], p['b1'])
    return _ffn2_ln(h1, p['W2'], p['b2'], x1, p['g2'], p['be2'])


def _layer_last8(x, p, cnt_t):
    ctx = _attention_ctx(x, p, cnt_t)
    ctx8 = ctx[:, _L - 8:, :].transpose(1, 0, 2).reshape(8, _DM)
    x8 = x[_L - 8:]
    attn8 = _matmul(ctx8, p['Wo'], p['bo'], bm=8)
    x1 = _addln(x8, attn8, p['g1'], p['be1'])
    h1 = _matmul(x1, p['W1'], p['b1'], act="gelu", bm=8)
    h2 = _matmul(h1, p['W2'], p['b2'], bm=8)
    return _addln(x1, h2, p['g2'], p['be2'])


def kernel(x_state_enc, x_enc, x_mark_enc, params):
    p = params
    st = x_state_enc.reshape(_L, 1).astype(jnp.int32)
    xe = x_enc.reshape(_L, 1).astype(jnp.float32)
    xm = x_mark_enc.reshape(_L, 4).astype(jnp.float32)

    bias0 = (p['b_val'] + p['b_mark']).reshape(1, _DM)
    pe = jnp.asarray(_PE)

    gathered = _sc_gather_rows(p['state_emb'], st.reshape(_L))
    x = _embed(gathered, xe, xm, pe, p['W_val'], p['W_mark'], bias0)
    x = _layer_full(x, p['layers'][0], jnp.asarray(_CNT_T[0]))
    x8 = _layer_last8(x, p['layers'][1], jnp.asarray(_CNT_T[1]))
    y = _final(x8, p['norm_g'], p['norm_b'], p['state_emb'].T)
    return y[7:8, :_NSTATE - 1].reshape(1, 1, _NSTATE - 1)
